# Initial kernel scaffold; baseline (speedup 1.0000x reference)
import jax
import jax.numpy as jnp
from jax.experimental import pallas as pl


def kernel(pos, z, batch, edge_index, params):
    raise NotImplementedError("write your pallas kernel here")


# speedup vs baseline: 1.3878x; 1.3878x over previous
import functools

import jax
import jax.numpy as jnp
from jax import lax
from jax.experimental import pallas as pl
from jax.experimental.pallas import tpu as pltpu
from jax.experimental.pallas import tpu_sc as plsc

f32 = jnp.float32
i32 = jnp.int32

CUT = 5.0
NSC = 2
NTILE = 16
NW = NSC * NTILE
CH = 64
LANES = 16


def _ssp(x):
    return jnp.maximum(x, 0.0) + jnp.log(1.0 + jnp.exp(-jnp.abs(x))) - 0.6931471805599453


def _sig(x):
    return 1.0 / (1.0 + jnp.exp(-x))


def _mesh():
    return plsc.VectorSubcoreMesh(core_axis_name="c", subcore_axis_name="s")


def _wid():
    return lax.axis_index("c") * NTILE + lax.axis_index("s")


def _mul_inplace(dst, a, b, rows, cols):
    @pl.loop(0, rows)
    def _(r):
        for k in range(cols // LANES):
            sl = pl.ds(k * LANES, LANES)
            dst[r, sl] = a[r, sl] * b[r, sl]


def _zero_buf(buf, rows, cols):
    @pl.loop(0, rows)
    def _(r):
        for k in range(cols // LANES):
            buf[r, pl.ds(k * LANES, LANES)] = jnp.zeros((LANES,), f32)


def _zero_shared(zbuf, shared, rows_per_tile):
    s = lax.axis_index("s")
    for j in range(rows_per_tile // CH):
        pltpu.sync_copy(zbuf, shared.at[pl.ds(s * rows_per_tile + j * CH, CH)])


def _sc_gather(table_shape, n_idx, chunk, tc_tiling=True):
    T, D = table_shape
    per_tile = n_idx // NW // chunk

    @functools.partial(
        pl.kernel, mesh=_mesh(),
        out_type=jax.ShapeDtypeStruct((n_idx, D), f32),
        compiler_params=pltpu.CompilerParams(use_tc_tiling_on_sc=tc_tiling),
        scratch_types=[
            pltpu.VMEM((chunk,), i32),
            pltpu.VMEM((chunk, D), f32),
            pltpu.SemaphoreType.DMA,
        ],
    )
    def k(table_hbm, idx_hbm, out_hbm, idxb, rows, sem):
        w = _wid()

        @pl.loop(0, per_tile)
        def _(j):
            base = (w * per_tile + j) * chunk
            pltpu.sync_copy(idx_hbm.at[pl.ds(base, chunk)], idxb)
            pltpu.async_copy(table_hbm.at[idxb], rows, sem).wait()
            pltpu.sync_copy(rows, out_hbm.at[pl.ds(base, chunk)])

    return k


def _sc_msg_fwd(NT, EP):
    per_tile = EP // NW // CH

    @functools.partial(
        pl.kernel, mesh=_mesh(),
        out_type=jax.ShapeDtypeStruct((NSC, NT, 128), f32),
        compiler_params=pltpu.CompilerParams(use_tc_tiling_on_sc=False),
        scratch_types=[
            pltpu.VMEM((CH,), i32),
            pltpu.VMEM((CH,), i32),
            pltpu.VMEM((CH, 128), f32),
            pltpu.VMEM((CH, 128), f32),
            pltpu.VMEM_SHARED((NT, 128), f32),
            pltpu.SemaphoreType.DMA,
        ],
    )
    def k(hx_hbm, w_hbm, row_hbm, col_hbm, out_hbm, rowi, coli, xj, wv, aggS, sem):
        c = lax.axis_index("c")
        s = lax.axis_index("s")
        w = c * NTILE + s
        rpt = NT // NTILE
        _zero_buf(wv, CH, 128)
        _zero_shared(wv, aggS, rpt)
        plsc.subcore_barrier()

        @pl.loop(0, per_tile)
        def _(j):
            base = (w * per_tile + j) * CH
            pltpu.sync_copy(row_hbm.at[pl.ds(base, CH)], rowi)
            pltpu.sync_copy(col_hbm.at[pl.ds(base, CH)], coli)
            pltpu.async_copy(hx_hbm.at[rowi], xj, sem).wait()
            pltpu.sync_copy(w_hbm.at[pl.ds(base, CH)], wv)
            _mul_inplace(wv, xj, wv, CH, 128)
            pltpu.sync_copy(wv, aggS.at[coli], add=True)

        plsc.subcore_barrier()
        pltpu.sync_copy(aggS.at[pl.ds(s * rpt, rpt)],
                        out_hbm.at[c, pl.ds(s * rpt, rpt)])

    return k


def _sc_msg_bwd(NT, EP):
    per_tile = EP // NW // CH

    @functools.partial(
        pl.kernel, mesh=_mesh(),
        out_type=(jax.ShapeDtypeStruct((EP, 128), f32),
                  jax.ShapeDtypeStruct((NSC, NT, 128), f32)),
        compiler_params=pltpu.CompilerParams(use_tc_tiling_on_sc=False),
        scratch_types=[
            pltpu.VMEM((CH,), i32),
            pltpu.VMEM((CH,), i32),
            pltpu.VMEM((CH, 128), f32),
            pltpu.VMEM((CH, 128), f32),
            pltpu.VMEM((CH, 128), f32),
            pltpu.VMEM_SHARED((NT, 128), f32),
            pltpu.SemaphoreType.DMA,
        ],
    )
    def k(dagg_hbm, hx_hbm, w_hbm, row_hbm, col_hbm, dw_hbm, dhx_hbm,
          rowi, coli, gv, xj, wv, dhxS, sem):
        c = lax.axis_index("c")
        s = lax.axis_index("s")
        w = c * NTILE + s
        rpt = NT // NTILE
        _zero_buf(wv, CH, 128)
        _zero_shared(wv, dhxS, rpt)
        plsc.subcore_barrier()

        @pl.loop(0, per_tile)
        def _(j):
            base = (w * per_tile + j) * CH
            pltpu.sync_copy(row_hbm.at[pl.ds(base, CH)], rowi)
            pltpu.sync_copy(col_hbm.at[pl.ds(base, CH)], coli)
            pltpu.async_copy(dagg_hbm.at[coli], gv, sem).wait()
            pltpu.async_copy(hx_hbm.at[rowi], xj, sem).wait()
            pltpu.sync_copy(w_hbm.at[pl.ds(base, CH)], wv)
            _mul_inplace(xj, gv, xj, CH, 128)
            _mul_inplace(wv, gv, wv, CH, 128)
            pltpu.sync_copy(xj, dw_hbm.at[pl.ds(base, CH)])
            pltpu.sync_copy(wv, dhxS.at[rowi], add=True)

        plsc.subcore_barrier()
        pltpu.sync_copy(dhxS.at[pl.ds(s * rpt, rpt)],
                        dhx_hbm.at[c, pl.ds(s * rpt, rpt)])

    return k


def _sc_scatter_force(NT, EP):
    per_tile = EP // NW // CH

    @functools.partial(
        pl.kernel, mesh=_mesh(),
        out_type=jax.ShapeDtypeStruct((NSC, 2, NT, 8), f32),
        compiler_params=pltpu.CompilerParams(use_tc_tiling_on_sc=False),
        scratch_types=[
            pltpu.VMEM((CH,), i32),
            pltpu.VMEM((CH,), i32),
            pltpu.VMEM((CH, 8), f32),
            pltpu.VMEM_SHARED((NT, 8), f32),
            pltpu.VMEM_SHARED((NT, 8), f32),
            pltpu.SemaphoreType.DMA,
        ],
    )
    def k(vec_hbm, row_hbm, col_hbm, out_hbm, rowi, coli, vv, frS, fcS, sem):
        c = lax.axis_index("c")
        s = lax.axis_index("s")
        w = c * NTILE + s
        rpt = NT // NTILE
        _zero_buf(vv, CH, 8)
        _zero_shared(vv, frS, rpt)
        _zero_shared(vv, fcS, rpt)
        plsc.subcore_barrier()

        @pl.loop(0, per_tile)
        def _(j):
            base = (w * per_tile + j) * CH
            pltpu.sync_copy(row_hbm.at[pl.ds(base, CH)], rowi)
            pltpu.sync_copy(col_hbm.at[pl.ds(base, CH)], coli)
            pltpu.sync_copy(vec_hbm.at[pl.ds(base, CH)], vv)
            pltpu.sync_copy(vv, frS.at[rowi], add=True)
            pltpu.sync_copy(vv, fcS.at[coli], add=True)

        plsc.subcore_barrier()
        pltpu.sync_copy(frS.at[pl.ds(s * rpt, rpt)],
                        out_hbm.at[c, 0, pl.ds(s * rpt, rpt)])
        pltpu.sync_copy(fcS.at[pl.ds(s * rpt, rpt)],
                        out_hbm.at[c, 1, pl.ds(s * rpt, rpt)])

    return k


def _tc_call(body, grid, in_specs, out_specs, out_shape):
    return pl.pallas_call(
        body, grid=grid, in_specs=in_specs, out_specs=out_specs,
        out_shape=out_shape)


def _edge_geom(EP, TE=1024):
    def body(pr_ref, pc_ref, d_ref, ew_ref):
        d = pr_ref[...] - pc_ref[...]
        d_ref[...] = d
        ew_ref[...] = jnp.sqrt(jnp.sum(d * d, axis=1, keepdims=True) + 1e-12)

    return _tc_call(
        body, (EP // TE,),
        [pl.BlockSpec((TE, 8), lambda i: (i, 0))] * 2,
        [pl.BlockSpec((TE, 8), lambda i: (i, 0)),
         pl.BlockSpec((TE, 1), lambda i: (i, 0))],
        [jax.ShapeDtypeStruct((EP, 8), f32),
         jax.ShapeDtypeStruct((EP, 1), f32)])


def _smear(ew, G, GP):
    delta = CUT / (G - 1)
    coeff = -0.5 / delta ** 2
    off = lax.broadcasted_iota(i32, (ew.shape[0], GP), 1).astype(f32) * delta
    return jnp.exp(coeff * (ew - off) ** 2), off, coeff


def _edge_w_fwd(EP, G, GP, TE=1024):
    def body(ew_ref, w1pT_ref, w2T_ref, bias_ref, wout_ref):
        ew = ew_ref[...]
        ea, _, _ = _smear(ew, G, GP)
        A1 = jnp.dot(ea, w1pT_ref[...], preferred_element_type=f32) + bias_ref[0:1, :]
        S1 = _ssp(A1)
        W0 = jnp.dot(S1, w2T_ref[...], preferred_element_type=f32) + bias_ref[1:2, :]
        C = 0.5 * (jnp.cos(ew * (jnp.pi / CUT)) + 1.0)
        wout_ref[...] = W0 * C

    return _tc_call(
        body, (EP // TE,),
        [pl.BlockSpec((TE, 1), lambda i: (i, 0)),
         pl.BlockSpec((GP, 128), lambda i: (0, 0)),
         pl.BlockSpec((128, 128), lambda i: (0, 0)),
         pl.BlockSpec((8, 128), lambda i: (0, 0))],
        pl.BlockSpec((TE, 128), lambda i: (i, 0)),
        jax.ShapeDtypeStruct((EP, 128), f32))


def _edge_w_bwd(EP, G, GP, TE=1024):
    def body(ew_ref, dW_ref, dewin_ref, w1pT_ref, w1p_ref, w2T_ref, w2_ref,
             bias_ref, dewout_ref):
        ew = ew_ref[...]
        ea, off, coeff = _smear(ew, G, GP)
        A1 = jnp.dot(ea, w1pT_ref[...], preferred_element_type=f32) + bias_ref[0:1, :]
        S1 = _ssp(A1)
        W0 = jnp.dot(S1, w2T_ref[...], preferred_element_type=f32) + bias_ref[1:2, :]
        dW = dW_ref[...]
        C = 0.5 * (jnp.cos(ew * (jnp.pi / CUT)) + 1.0)
        dC = jnp.sum(dW * W0, axis=1, keepdims=True)
        dW0 = dW * C
        dS1 = jnp.dot(dW0, w2_ref[...], preferred_element_type=f32)
        dA1 = dS1 * _sig(A1)
        dea = jnp.dot(dA1, w1p_ref[...], preferred_element_type=f32)
        dea_dew = ea * (2.0 * coeff) * (ew - off)
        dCdew = -0.5 * jnp.sin(ew * (jnp.pi / CUT)) * (jnp.pi / CUT)
        dewout_ref[...] = (dewin_ref[...] + jnp.sum(dea * dea_dew, axis=1, keepdims=True)
                           + dC * dCdew)

    return _tc_call(
        body, (EP // TE,),
        [pl.BlockSpec((TE, 1), lambda i: (i, 0)),
         pl.BlockSpec((TE, 128), lambda i: (i, 0)),
         pl.BlockSpec((TE, 1), lambda i: (i, 0)),
         pl.BlockSpec((GP, 128), lambda i: (0, 0)),
         pl.BlockSpec((128, GP), lambda i: (0, 0)),
         pl.BlockSpec((128, 128), lambda i: (0, 0)),
         pl.BlockSpec((128, 128), lambda i: (0, 0)),
         pl.BlockSpec((8, 128), lambda i: (0, 0))],
        pl.BlockSpec((TE, 1), lambda i: (i, 0)),
        jax.ShapeDtypeStruct((EP, 1), f32))


def _node_mm(NT, TB=512):
    def body(x_ref, wT_ref, out_ref):
        out_ref[...] = jnp.dot(x_ref[...], wT_ref[...], preferred_element_type=f32)

    return _tc_call(
        body, (NT // TB,),
        [pl.BlockSpec((TB, 128), lambda i: (i, 0)),
         pl.BlockSpec((128, 128), lambda i: (0, 0))],
        pl.BlockSpec((TB, 128), lambda i: (i, 0)),
        jax.ShapeDtypeStruct((NT, 128), f32))


def _node_fwd(NT, TB=512):
    def body(h_ref, a0_ref, a1_ref, lin2T_ref, linT_ref, bias_ref,
             hout_ref, agg_ref):
        agg = a0_ref[...] + a1_ref[...]
        agg_ref[...] = agg
        A2 = jnp.dot(agg, lin2T_ref[...], preferred_element_type=f32) + bias_ref[2:3, :]
        S2 = _ssp(A2)
        hc = jnp.dot(S2, linT_ref[...], preferred_element_type=f32) + bias_ref[3:4, :]
        hout_ref[...] = h_ref[...] + hc

    return _tc_call(
        body, (NT // TB,),
        [pl.BlockSpec((TB, 128), lambda i: (i, 0))] * 3 +
        [pl.BlockSpec((128, 128), lambda i: (0, 0))] * 2 +
        [pl.BlockSpec((8, 128), lambda i: (0, 0))],
        [pl.BlockSpec((TB, 128), lambda i: (i, 0))] * 2,
        [jax.ShapeDtypeStruct((NT, 128), f32)] * 2)


def _node_bwd1(NT, TB=512):
    def body(dh_ref, agg_ref, lin_ref, lin2T_ref, lin2_ref, bias_ref, dagg_ref):
        dS2 = jnp.dot(dh_ref[...], lin_ref[...], preferred_element_type=f32)
        A2 = jnp.dot(agg_ref[...], lin2T_ref[...], preferred_element_type=f32) + bias_ref[2:3, :]
        dA2 = dS2 * _sig(A2)
        dagg_ref[...] = jnp.dot(dA2, lin2_ref[...], preferred_element_type=f32)

    return _tc_call(
        body, (NT // TB,),
        [pl.BlockSpec((TB, 128), lambda i: (i, 0))] * 2 +
        [pl.BlockSpec((128, 128), lambda i: (0, 0))] * 3 +
        [pl.BlockSpec((8, 128), lambda i: (0, 0))],
        pl.BlockSpec((TB, 128), lambda i: (i, 0)),
        jax.ShapeDtypeStruct((NT, 128), f32))


def _node_bwd2(NT, TB=512):
    def body(dh_ref, d0_ref, d1_ref, lin1_ref, out_ref):
        dhx = d0_ref[...] + d1_ref[...]
        out_ref[...] = dh_ref[...] + jnp.dot(dhx, lin1_ref[...],
                                             preferred_element_type=f32)

    return _tc_call(
        body, (NT // TB,),
        [pl.BlockSpec((TB, 128), lambda i: (i, 0))] * 3 +
        [pl.BlockSpec((128, 128), lambda i: (0, 0))],
        pl.BlockSpec((TB, 128), lambda i: (i, 0)),
        jax.ShapeDtypeStruct((NT, 128), f32))


def _readout_fwd(NT, TB=512):
    def body(h_ref, batch_ref, w1rT_ref, small_ref, out_ref):
        i = pl.program_id(0)
        A3 = jnp.dot(h_ref[...], w1rT_ref[...], preferred_element_type=f32) + small_ref[0:1, :]
        S3 = _ssp(A3)
        y = jnp.sum(S3 * small_ref[1:2, :], axis=1, keepdims=True) + small_ref[2:3, 0:1]
        b = batch_ref[...]
        onehot = (b == lax.broadcasted_iota(i32, (TB, 64), 1)).astype(f32)
        contrib = jnp.sum(y * onehot, axis=0, keepdims=True)

        @pl.when(i == 0)
        def _():
            out_ref[...] = jnp.zeros_like(out_ref)

        out_ref[0:1, :] = out_ref[0:1, :] + contrib

    return _tc_call(
        body, (NT // TB,),
        [pl.BlockSpec((TB, 128), lambda i: (i, 0)),
         pl.BlockSpec((TB, 1), lambda i: (i, 0)),
         pl.BlockSpec((128, 64), lambda i: (0, 0)),
         pl.BlockSpec((8, 64), lambda i: (0, 0))],
        pl.BlockSpec((8, 64), lambda i: (0, 0)),
        jax.ShapeDtypeStruct((8, 64), f32))


def _readout_bwd(NT, TB=512):
    def body(h_ref, w1rT_ref, w1r_ref, small_ref, dh_ref):
        A3 = jnp.dot(h_ref[...], w1rT_ref[...], preferred_element_type=f32) + small_ref[0:1, :]
        dA3 = _sig(A3) * small_ref[1:2, :]
        dh_ref[...] = jnp.dot(dA3, w1r_ref[...], preferred_element_type=f32)

    return _tc_call(
        body, (NT // TB,),
        [pl.BlockSpec((TB, 128), lambda i: (i, 0)),
         pl.BlockSpec((128, 64), lambda i: (0, 0)),
         pl.BlockSpec((64, 128), lambda i: (0, 0)),
         pl.BlockSpec((8, 64), lambda i: (0, 0))],
        pl.BlockSpec((TB, 128), lambda i: (i, 0)),
        jax.ShapeDtypeStruct((NT, 128), f32))


def _force_vec(EP, TE=1024):
    def body(dew_ref, ew_ref, d_ref, out_ref):
        out_ref[...] = (dew_ref[...] / ew_ref[...]) * d_ref[...]

    return _tc_call(
        body, (EP // TE,),
        [pl.BlockSpec((TE, 1), lambda i: (i, 0)),
         pl.BlockSpec((TE, 1), lambda i: (i, 0)),
         pl.BlockSpec((TE, 8), lambda i: (i, 0))],
        pl.BlockSpec((TE, 8), lambda i: (i, 0)),
        jax.ShapeDtypeStruct((EP, 8), f32))


def _force_combine(NT, TB=512):
    def body(r0_ref, r1_ref, c0_ref, c1_ref, out_ref):
        out_ref[...] = (c0_ref[...] + c1_ref[...]) - (r0_ref[...] + r1_ref[...])

    return _tc_call(
        body, (NT // TB,),
        [pl.BlockSpec((TB, 8), lambda i: (i, 0))] * 4,
        pl.BlockSpec((TB, 8), lambda i: (i, 0)),
        jax.ShapeDtypeStruct((NT, 8), f32))


def kernel(pos, z, batch, edge_index, params):
    N = pos.shape[0]
    E = edge_index.shape[1]
    MAXZ, H = params['emb'].shape
    G = params['blocks'][0]['mlp_w1'].shape[1]
    GP = 64

    NT = ((N + 1 + 2047) // 2048) * 2048
    EP = ((E + NW * CH - 1) // (NW * CH)) * (NW * CH)

    row = edge_index[0].astype(i32)
    col = edge_index[1].astype(i32)
    rowp = jnp.concatenate([row, jnp.full((EP - E,), N, i32)])
    colp = jnp.concatenate([col, jnp.full((EP - E,), N, i32)])
    pos8 = jnp.zeros((NT, 8), f32).at[:N, :3].set(pos.astype(f32))
    zp = jnp.zeros((NT,), i32).at[:N].set(z.astype(i32))
    batchp = jnp.full((NT, 1), jnp.int32(1 << 20)).at[:N, 0].set(batch.astype(i32))

    blocks = params['blocks']
    wT = []
    for blk in blocks:
        w1p = jnp.zeros((128, GP), f32).at[:, :G].set(blk['mlp_w1'])
        bias = jnp.zeros((8, 128), f32)
        bias = bias.at[0, :].set(blk['mlp_b1']).at[1, :].set(blk['mlp_b2'])
        bias = bias.at[2, :].set(blk['lin2_b']).at[3, :].set(blk['lin_b'])
        wT.append(dict(
            w1pT=w1p.T, w1p=w1p, w2T=blk['mlp_w2'].T, w2=blk['mlp_w2'],
            lin1T=blk['lin1_w'].T, lin1=blk['lin1_w'],
            lin2T=blk['lin2_w'].T, lin2=blk['lin2_w'],
            linT=blk['lin_w'].T, lin=blk['lin_w'], bias=bias))
    w1rT = params['w1'].T
    w1r = params['w1']
    small = jnp.zeros((8, 64), f32)
    small = small.at[0, :].set(params['b1']).at[1, :].set(params['w2'][0])
    small = small.at[2, :].set(jnp.broadcast_to(params['b2'], (64,)))

    gather_pos = _sc_gather((NT, 8), EP, CH, tc_tiling=False)
    gather_emb = _sc_gather((MAXZ, H), NT, 64)
    msg_fwd = _sc_msg_fwd(NT, EP)
    msg_bwd = _sc_msg_bwd(NT, EP)
    scat_force = _sc_scatter_force(NT, EP)
    edge_geom = _edge_geom(EP)
    ew_fwd = _edge_w_fwd(EP, G, GP)
    ew_bwd = _edge_w_bwd(EP, G, GP)
    node_mm = _node_mm(NT)
    node_fwd = _node_fwd(NT)
    node_bwd1 = _node_bwd1(NT)
    node_bwd2 = _node_bwd2(NT)
    ro_fwd = _readout_fwd(NT)
    ro_bwd = _readout_bwd(NT)
    fvec = _force_vec(EP)
    fcomb = _force_combine(NT)

    prow = gather_pos(pos8, rowp)
    pcol = gather_pos(pos8, colp)
    d8, ew = edge_geom(prow, pcol)
    h = gather_emb(params['emb'], zp)

    hs, hxs, aggs, Ws = [], [], [], []
    for bi in range(len(blocks)):
        t = wT[bi]
        hs.append(h)
        hx = node_mm(h, t['lin1T'])
        hxs.append(hx)
        W = ew_fwd(ew, t['w1pT'], t['w2T'], t['bias'])
        Ws.append(W)
        aggpair = msg_fwd(hx, W, rowp, colp)
        h, agg = node_fwd(h, aggpair[0], aggpair[1], t['lin2T'], t['linT'],
                          t['bias'])
        aggs.append(agg)

    out8 = ro_fwd(h, batchp, w1rT, small)
    out = out8[0]

    dh = ro_bwd(h, w1rT, w1r, small)
    dew = jnp.zeros((EP, 1), f32)
    for bi in reversed(range(len(blocks))):
        t = wT[bi]
        dagg = node_bwd1(dh, aggs[bi], t['lin'], t['lin2T'], t['lin2'],
                         t['bias'])
        dW, dhxpair = msg_bwd(dagg, hxs[bi], Ws[bi], rowp, colp)
        dh = node_bwd2(dh, dhxpair[0], dhxpair[1], t['lin1'])
        dew = ew_bwd(ew, dW, dew, t['w1pT'], t['w1p'], t['w2T'], t['w2'],
                     t['bias'])

    vec = fvec(dew, ew, d8)
    fparts = scat_force(vec, rowp, colp)
    fneg = fcomb(fparts[0, 0], fparts[1, 0], fparts[0, 1], fparts[1, 1])
    forces = fneg[:N, :3]
    return out, forces


# speedup vs baseline: 1.7280x; 1.2452x over previous
import functools
import math

import jax
import jax.numpy as jnp
from jax import lax
from jax.experimental import pallas as pl
from jax.experimental.pallas import tpu as pltpu
from jax.experimental.pallas import tpu_sc as plsc

f32 = jnp.float32
i32 = jnp.int32

CUT = 5.0
NSC = 2
NTILE = 16
NW = NSC * NTILE
CH = 128
LANES = 16


def _ssp(x):
    return jnp.maximum(x, 0.0) + jnp.log(1.0 + jnp.exp(-jnp.abs(x))) - 0.6931471805599453


def _sig(x):
    return 1.0 / (1.0 + jnp.exp(-x))


def _mesh():
    return plsc.VectorSubcoreMesh(core_axis_name="c", subcore_axis_name="s")


def _wid():
    return lax.axis_index("c") * NTILE + lax.axis_index("s")


def _mul_inplace(dst, a, b, rows, cols):
    @pl.loop(0, rows)
    def _(r):
        for k in range(cols // LANES):
            sl = pl.ds(k * LANES, LANES)
            dst[r, sl] = a[r, sl] * b[r, sl]


def _zero_buf(buf, rows, cols):
    @pl.loop(0, rows)
    def _(r):
        for k in range(cols // LANES):
            buf[r, pl.ds(k * LANES, LANES)] = jnp.zeros((LANES,), f32)


def _zero_shared(zbuf, shared, rows_per_tile, chunk):
    s = lax.axis_index("s")
    nfull, rem = rows_per_tile // chunk, rows_per_tile % chunk
    for j in range(nfull):
        pltpu.sync_copy(zbuf, shared.at[pl.ds(s * rows_per_tile + j * chunk, chunk)])
    if rem:
        pltpu.sync_copy(zbuf.at[pl.ds(0, rem)],
                        shared.at[pl.ds(s * rows_per_tile + nfull * chunk, rem)])


def _stage_idx(idx2_hbm, idx2_v, w, per_tile):
    pltpu.sync_copy(idx2_hbm.at[pl.ds(w * per_tile, per_tile)], idx2_v)


def _sc_gather(table_shape, n_idx, chunk):
    T, D = table_shape
    per_tile = n_idx // NW // chunk
    assert per_tile % 2 == 0

    @functools.partial(
        pl.kernel, mesh=_mesh(),
        out_type=jax.ShapeDtypeStruct((n_idx, D), f32),
        compiler_params=pltpu.CompilerParams(use_tc_tiling_on_sc=False),
        scratch_types=[
            pltpu.VMEM((per_tile, chunk), i32),
            pltpu.VMEM((chunk, D), f32),
            pltpu.VMEM((chunk, D), f32),
            pltpu.SemaphoreType.DMA,
            pltpu.SemaphoreType.DMA,
        ],
    )
    def k(table_hbm, idx2_hbm, out_hbm, idx2, r0, r1, sg0, sg1):
        w = _wid()
        rows = (r0, r1)
        sg = (sg0, sg1)
        _stage_idx(idx2_hbm, idx2, w, per_tile)

        def issue(j, b):
            pltpu.async_copy(table_hbm.at[idx2.at[j]], rows[b], sg[b])

        issue(0, 0)
        issue(1, 1)

        @pl.loop(0, per_tile, step=2)
        def _(g):
            for b in (0, 1):
                j = g + b
                base = (w * per_tile + j) * chunk
                pltpu.make_async_copy(table_hbm.at[idx2.at[j]], rows[b], sg[b]).wait()
                pltpu.sync_copy(rows[b], out_hbm.at[pl.ds(base, chunk)])

                @pl.when(j + 2 < per_tile)
                def _():
                    issue(j + 2, b)

    return k


def _sc_msg_fwd(NT, NTS, EP, CH):
    per_tile = EP // NW // CH
    assert per_tile % 2 == 0

    @functools.partial(
        pl.kernel, mesh=_mesh(),
        out_type=jax.ShapeDtypeStruct((NSC, NT, 128), f32),
        compiler_params=pltpu.CompilerParams(use_tc_tiling_on_sc=False),
        scratch_types=[
            pltpu.VMEM((per_tile, CH), i32),
            pltpu.VMEM((per_tile, CH), i32),
            pltpu.VMEM((CH, 128), f32),
            pltpu.VMEM((CH, 128), f32),
            pltpu.VMEM((CH, 128), f32),
            pltpu.VMEM((CH, 128), f32),
            pltpu.VMEM_SHARED((NTS, 128), f32),
            pltpu.SemaphoreType.DMA,
            pltpu.SemaphoreType.DMA,
            pltpu.SemaphoreType.DMA,
            pltpu.SemaphoreType.DMA,
        ],
    )
    def k(hx_hbm, w_hbm, row2_hbm, col2_hbm, out_hbm, ridx, cidx,
          xj0, xj1, wv0, wv1, aggS, sg0, sg1, sw0, sw1):
        c = lax.axis_index("c")
        s = lax.axis_index("s")
        w = c * NTILE + s
        rpt = NTS // NTILE
        xj = (xj0, xj1)
        wv = (wv0, wv1)
        sg = (sg0, sg1)
        sw = (sw0, sw1)
        _zero_buf(wv0, CH, 128)
        _zero_shared(wv0, aggS, rpt, CH)
        _stage_idx(row2_hbm, ridx, w, per_tile)
        _stage_idx(col2_hbm, cidx, w, per_tile)
        plsc.subcore_barrier()

        def issue(j, b):
            base = (w * per_tile + j) * CH
            pltpu.async_copy(hx_hbm.at[ridx.at[j]], xj[b], sg[b])
            pltpu.async_copy(w_hbm.at[pl.ds(base, CH)], wv[b], sw[b])

        issue(0, 0)
        issue(1, 1)

        @pl.loop(0, per_tile, step=2)
        def _(g):
            for b in (0, 1):
                j = g + b
                pltpu.make_async_copy(hx_hbm.at[ridx.at[j]], xj[b], sg[b]).wait()
                pltpu.make_async_copy(w_hbm.at[pl.ds(0, CH)], wv[b], sw[b]).wait()
                _mul_inplace(wv[b], xj[b], wv[b], CH, 128)
                pltpu.sync_copy(wv[b], aggS.at[cidx.at[j]], add=True)

                @pl.when(j + 2 < per_tile)
                def _():
                    issue(j + 2, b)

        plsc.subcore_barrier()
        pltpu.sync_copy(aggS.at[pl.ds(s * rpt, rpt)],
                        out_hbm.at[c, pl.ds(s * rpt, rpt)])

    return k


def _sc_msg_bwd(NT, NTS, EP, CH):
    per_tile = EP // NW // CH
    assert per_tile % 2 == 0

    @functools.partial(
        pl.kernel, mesh=_mesh(),
        out_type=(jax.ShapeDtypeStruct((EP, 128), f32),
                  jax.ShapeDtypeStruct((NSC, NT, 128), f32)),
        compiler_params=pltpu.CompilerParams(use_tc_tiling_on_sc=False),
        scratch_types=[
            pltpu.VMEM((per_tile, CH), i32),
            pltpu.VMEM((per_tile, CH), i32),
            pltpu.VMEM((CH, 128), f32),
            pltpu.VMEM((CH, 128), f32),
            pltpu.VMEM((CH, 128), f32),
            pltpu.VMEM((CH, 128), f32),
            pltpu.VMEM((CH, 128), f32),
            pltpu.VMEM((CH, 128), f32),
            pltpu.VMEM_SHARED((NTS, 128), f32),
            pltpu.SemaphoreType.DMA,
            pltpu.SemaphoreType.DMA,
            pltpu.SemaphoreType.DMA,
            pltpu.SemaphoreType.DMA,
            pltpu.SemaphoreType.DMA,
            pltpu.SemaphoreType.DMA,
        ],
    )
    def k(dagg_hbm, hx_hbm, w_hbm, row2_hbm, col2_hbm, dw_hbm, dhx_hbm,
          ridx, cidx, gv0, gv1, xj0, xj1, wv0, wv1, dhxS,
          sa0, sa1, sx0, sx1, sw0, sw1):
        c = lax.axis_index("c")
        s = lax.axis_index("s")
        w = c * NTILE + s
        rpt = NTS // NTILE
        gv = (gv0, gv1)
        xj = (xj0, xj1)
        wv = (wv0, wv1)
        sa = (sa0, sa1)
        sx = (sx0, sx1)
        sw = (sw0, sw1)
        _zero_buf(wv0, CH, 128)
        _zero_shared(wv0, dhxS, rpt, CH)
        _stage_idx(row2_hbm, ridx, w, per_tile)
        _stage_idx(col2_hbm, cidx, w, per_tile)
        plsc.subcore_barrier()

        def issue(j, b):
            base = (w * per_tile + j) * CH
            pltpu.async_copy(dagg_hbm.at[cidx.at[j]], gv[b], sa[b])
            pltpu.async_copy(hx_hbm.at[ridx.at[j]], xj[b], sx[b])
            pltpu.async_copy(w_hbm.at[pl.ds(base, CH)], wv[b], sw[b])

        issue(0, 0)
        issue(1, 1)

        @pl.loop(0, per_tile, step=2)
        def _(g):
            for b in (0, 1):
                j = g + b
                base = (w * per_tile + j) * CH
                pltpu.make_async_copy(dagg_hbm.at[cidx.at[j]], gv[b], sa[b]).wait()
                pltpu.make_async_copy(hx_hbm.at[ridx.at[j]], xj[b], sx[b]).wait()
                pltpu.make_async_copy(w_hbm.at[pl.ds(0, CH)], wv[b], sw[b]).wait()
                _mul_inplace(xj[b], gv[b], xj[b], CH, 128)
                _mul_inplace(wv[b], gv[b], wv[b], CH, 128)
                pltpu.sync_copy(xj[b], dw_hbm.at[pl.ds(base, CH)])
                pltpu.sync_copy(wv[b], dhxS.at[ridx.at[j]], add=True)

                @pl.when(j + 2 < per_tile)
                def _():
                    issue(j + 2, b)

        plsc.subcore_barrier()
        pltpu.sync_copy(dhxS.at[pl.ds(s * rpt, rpt)],
                        dhx_hbm.at[c, pl.ds(s * rpt, rpt)])

    return k


def _sc_scatter_force(NT, NTS, EP, CH):
    per_tile = EP // NW // CH
    assert per_tile % 2 == 0

    @functools.partial(
        pl.kernel, mesh=_mesh(),
        out_type=jax.ShapeDtypeStruct((NSC, 2, NT, 8), f32),
        compiler_params=pltpu.CompilerParams(use_tc_tiling_on_sc=False),
        scratch_types=[
            pltpu.VMEM((per_tile, CH), i32),
            pltpu.VMEM((per_tile, CH), i32),
            pltpu.VMEM((CH, 8), f32),
            pltpu.VMEM((CH, 8), f32),
            pltpu.VMEM_SHARED((NTS, 8), f32),
            pltpu.VMEM_SHARED((NTS, 8), f32),
            pltpu.SemaphoreType.DMA,
            pltpu.SemaphoreType.DMA,
        ],
    )
    def k(vec_hbm, row2_hbm, col2_hbm, out_hbm, ridx, cidx, v0, v1,
          frS, fcS, sv0, sv1):
        c = lax.axis_index("c")
        s = lax.axis_index("s")
        w = c * NTILE + s
        rpt = NTS // NTILE
        vv = (v0, v1)
        sv = (sv0, sv1)
        _zero_buf(v0, CH, 8)
        _zero_shared(v0, frS, rpt, CH)
        _zero_shared(v0, fcS, rpt, CH)
        _stage_idx(row2_hbm, ridx, w, per_tile)
        _stage_idx(col2_hbm, cidx, w, per_tile)
        plsc.subcore_barrier()

        def issue(j, b):
            base = (w * per_tile + j) * CH
            pltpu.async_copy(vec_hbm.at[pl.ds(base, CH)], vv[b], sv[b])

        issue(0, 0)
        issue(1, 1)

        @pl.loop(0, per_tile, step=2)
        def _(g):
            for b in (0, 1):
                j = g + b
                pltpu.make_async_copy(vec_hbm.at[pl.ds(0, CH)], vv[b], sv[b]).wait()
                pltpu.sync_copy(vv[b], frS.at[ridx.at[j]], add=True)
                pltpu.sync_copy(vv[b], fcS.at[cidx.at[j]], add=True)

                @pl.when(j + 2 < per_tile)
                def _():
                    issue(j + 2, b)

        plsc.subcore_barrier()
        pltpu.sync_copy(frS.at[pl.ds(s * rpt, rpt)],
                        out_hbm.at[c, 0, pl.ds(s * rpt, rpt)])
        pltpu.sync_copy(fcS.at[pl.ds(s * rpt, rpt)],
                        out_hbm.at[c, 1, pl.ds(s * rpt, rpt)])

    return k


def _tc_call(body, grid, in_specs, out_specs, out_shape):
    return pl.pallas_call(
        body, grid=grid, in_specs=in_specs, out_specs=out_specs,
        out_shape=out_shape)


def _edge_geom(EP, TE=1024):
    def body(pr_ref, pc_ref, d_ref, ew_ref):
        d = pr_ref[...] - pc_ref[...]
        d_ref[...] = d
        ew_ref[...] = jnp.sqrt(jnp.sum(d * d, axis=1, keepdims=True) + 1e-12)

    return _tc_call(
        body, (EP // TE,),
        [pl.BlockSpec((TE, 8), lambda i: (i, 0))] * 2,
        [pl.BlockSpec((TE, 8), lambda i: (i, 0)),
         pl.BlockSpec((TE, 1), lambda i: (i, 0))],
        [jax.ShapeDtypeStruct((EP, 8), f32),
         jax.ShapeDtypeStruct((EP, 1), f32)])


def _smear(ew, G, GP):
    delta = CUT / (G - 1)
    coeff = -0.5 / delta ** 2
    off = lax.broadcasted_iota(i32, (ew.shape[0], GP), 1).astype(f32) * delta
    return jnp.exp(coeff * (ew - off) ** 2), off, coeff


def _edge_w_fwd(EP, G, GP, TE=1024):
    def body(ew_ref, w1pT_ref, w2T_ref, bias_ref, wout_ref):
        ew = ew_ref[...]
        ea, _, _ = _smear(ew, G, GP)
        A1 = jnp.dot(ea, w1pT_ref[...], preferred_element_type=f32) + bias_ref[0:1, :]
        S1 = _ssp(A1)
        W0 = jnp.dot(S1, w2T_ref[...], preferred_element_type=f32) + bias_ref[1:2, :]
        C = 0.5 * (jnp.cos(ew * (jnp.pi / CUT)) + 1.0)
        wout_ref[...] = W0 * C

    return _tc_call(
        body, (EP // TE,),
        [pl.BlockSpec((TE, 1), lambda i: (i, 0)),
         pl.BlockSpec((GP, 128), lambda i: (0, 0)),
         pl.BlockSpec((128, 128), lambda i: (0, 0)),
         pl.BlockSpec((8, 128), lambda i: (0, 0))],
        pl.BlockSpec((TE, 128), lambda i: (i, 0)),
        jax.ShapeDtypeStruct((EP, 128), f32))


def _edge_w_bwd(EP, G, GP, TE=1024):
    def body(ew_ref, dW_ref, dewin_ref, w1pT_ref, w1p_ref, w2T_ref, w2_ref,
             bias_ref, dewout_ref):
        ew = ew_ref[...]
        ea, off, coeff = _smear(ew, G, GP)
        A1 = jnp.dot(ea, w1pT_ref[...], preferred_element_type=f32) + bias_ref[0:1, :]
        S1 = _ssp(A1)
        W0 = jnp.dot(S1, w2T_ref[...], preferred_element_type=f32) + bias_ref[1:2, :]
        dW = dW_ref[...]
        C = 0.5 * (jnp.cos(ew * (jnp.pi / CUT)) + 1.0)
        dC = jnp.sum(dW * W0, axis=1, keepdims=True)
        dW0 = dW * C
        dS1 = jnp.dot(dW0, w2_ref[...], preferred_element_type=f32)
        dA1 = dS1 * _sig(A1)
        dea = jnp.dot(dA1, w1p_ref[...], preferred_element_type=f32)
        dea_dew = ea * (2.0 * coeff) * (ew - off)
        dCdew = -0.5 * jnp.sin(ew * (jnp.pi / CUT)) * (jnp.pi / CUT)
        dewout_ref[...] = (dewin_ref[...] + jnp.sum(dea * dea_dew, axis=1, keepdims=True)
                           + dC * dCdew)

    return _tc_call(
        body, (EP // TE,),
        [pl.BlockSpec((TE, 1), lambda i: (i, 0)),
         pl.BlockSpec((TE, 128), lambda i: (i, 0)),
         pl.BlockSpec((TE, 1), lambda i: (i, 0)),
         pl.BlockSpec((GP, 128), lambda i: (0, 0)),
         pl.BlockSpec((128, GP), lambda i: (0, 0)),
         pl.BlockSpec((128, 128), lambda i: (0, 0)),
         pl.BlockSpec((128, 128), lambda i: (0, 0)),
         pl.BlockSpec((8, 128), lambda i: (0, 0))],
        pl.BlockSpec((TE, 1), lambda i: (i, 0)),
        jax.ShapeDtypeStruct((EP, 1), f32))


def _node_mm(NT, TB=512):
    def body(x_ref, wT_ref, out_ref):
        out_ref[...] = jnp.dot(x_ref[...], wT_ref[...], preferred_element_type=f32)

    return _tc_call(
        body, (NT // TB,),
        [pl.BlockSpec((TB, 128), lambda i: (i, 0)),
         pl.BlockSpec((128, 128), lambda i: (0, 0))],
        pl.BlockSpec((TB, 128), lambda i: (i, 0)),
        jax.ShapeDtypeStruct((NT, 128), f32))


def _node_fwd(NT, TB=512):
    def body(h_ref, a0_ref, a1_ref, lin2T_ref, linT_ref, bias_ref,
             hout_ref, agg_ref):
        agg = a0_ref[...] + a1_ref[...]
        agg_ref[...] = agg
        A2 = jnp.dot(agg, lin2T_ref[...], preferred_element_type=f32) + bias_ref[2:3, :]
        S2 = _ssp(A2)
        hc = jnp.dot(S2, linT_ref[...], preferred_element_type=f32) + bias_ref[3:4, :]
        hout_ref[...] = h_ref[...] + hc

    return _tc_call(
        body, (NT // TB,),
        [pl.BlockSpec((TB, 128), lambda i: (i, 0))] * 3 +
        [pl.BlockSpec((128, 128), lambda i: (0, 0))] * 2 +
        [pl.BlockSpec((8, 128), lambda i: (0, 0))],
        [pl.BlockSpec((TB, 128), lambda i: (i, 0))] * 2,
        [jax.ShapeDtypeStruct((NT, 128), f32)] * 2)


def _node_bwd1(NT, TB=512):
    def body(dh_ref, agg_ref, lin_ref, lin2T_ref, lin2_ref, bias_ref, dagg_ref):
        dS2 = jnp.dot(dh_ref[...], lin_ref[...], preferred_element_type=f32)
        A2 = jnp.dot(agg_ref[...], lin2T_ref[...], preferred_element_type=f32) + bias_ref[2:3, :]
        dA2 = dS2 * _sig(A2)
        dagg_ref[...] = jnp.dot(dA2, lin2_ref[...], preferred_element_type=f32)

    return _tc_call(
        body, (NT // TB,),
        [pl.BlockSpec((TB, 128), lambda i: (i, 0))] * 2 +
        [pl.BlockSpec((128, 128), lambda i: (0, 0))] * 3 +
        [pl.BlockSpec((8, 128), lambda i: (0, 0))],
        pl.BlockSpec((TB, 128), lambda i: (i, 0)),
        jax.ShapeDtypeStruct((NT, 128), f32))


def _node_bwd2(NT, TB=512):
    def body(dh_ref, d0_ref, d1_ref, lin1_ref, out_ref):
        dhx = d0_ref[...] + d1_ref[...]
        out_ref[...] = dh_ref[...] + jnp.dot(dhx, lin1_ref[...],
                                             preferred_element_type=f32)

    return _tc_call(
        body, (NT // TB,),
        [pl.BlockSpec((TB, 128), lambda i: (i, 0))] * 3 +
        [pl.BlockSpec((128, 128), lambda i: (0, 0))],
        pl.BlockSpec((TB, 128), lambda i: (i, 0)),
        jax.ShapeDtypeStruct((NT, 128), f32))


def _readout_fwd(NT, TB=512):
    def body(h_ref, batch_ref, w1rT_ref, small_ref, out_ref):
        i = pl.program_id(0)
        A3 = jnp.dot(h_ref[...], w1rT_ref[...], preferred_element_type=f32) + small_ref[0:1, :]
        S3 = _ssp(A3)
        y = jnp.sum(S3 * small_ref[1:2, :], axis=1, keepdims=True) + small_ref[2:3, 0:1]
        b = batch_ref[...]
        y = jnp.where(b < 64, y, 0.0)
        onehot = (b == lax.broadcasted_iota(i32, (TB, 64), 1)).astype(f32)
        contrib = jnp.sum(y * onehot, axis=0, keepdims=True)

        @pl.when(i == 0)
        def _():
            out_ref[...] = jnp.zeros_like(out_ref)

        out_ref[0:1, :] = out_ref[0:1, :] + contrib

    return _tc_call(
        body, (NT // TB,),
        [pl.BlockSpec((TB, 128), lambda i: (i, 0)),
         pl.BlockSpec((TB, 1), lambda i: (i, 0)),
         pl.BlockSpec((128, 64), lambda i: (0, 0)),
         pl.BlockSpec((8, 64), lambda i: (0, 0))],
        pl.BlockSpec((8, 64), lambda i: (0, 0)),
        jax.ShapeDtypeStruct((8, 64), f32))


def _readout_bwd(NT, TB=512):
    def body(h_ref, w1rT_ref, w1r_ref, small_ref, dh_ref):
        A3 = jnp.dot(h_ref[...], w1rT_ref[...], preferred_element_type=f32) + small_ref[0:1, :]
        dA3 = _sig(A3) * small_ref[1:2, :]
        dh_ref[...] = jnp.dot(dA3, w1r_ref[...], preferred_element_type=f32)

    return _tc_call(
        body, (NT // TB,),
        [pl.BlockSpec((TB, 128), lambda i: (i, 0)),
         pl.BlockSpec((128, 64), lambda i: (0, 0)),
         pl.BlockSpec((64, 128), lambda i: (0, 0)),
         pl.BlockSpec((8, 64), lambda i: (0, 0))],
        pl.BlockSpec((TB, 128), lambda i: (i, 0)),
        jax.ShapeDtypeStruct((NT, 128), f32))


def _force_vec(EP, TE=1024):
    def body(dew_ref, ew_ref, d_ref, out_ref):
        out_ref[...] = (dew_ref[...] / ew_ref[...]) * d_ref[...]

    return _tc_call(
        body, (EP // TE,),
        [pl.BlockSpec((TE, 1), lambda i: (i, 0)),
         pl.BlockSpec((TE, 1), lambda i: (i, 0)),
         pl.BlockSpec((TE, 8), lambda i: (i, 0))],
        pl.BlockSpec((TE, 8), lambda i: (i, 0)),
        jax.ShapeDtypeStruct((EP, 8), f32))


def _force_combine(NT, TB=512):
    def body(r0_ref, r1_ref, c0_ref, c1_ref, out_ref):
        out_ref[...] = (c0_ref[...] + c1_ref[...]) - (r0_ref[...] + r1_ref[...])

    return _tc_call(
        body, (NT // TB,),
        [pl.BlockSpec((TB, 8), lambda i: (i, 0))] * 4,
        pl.BlockSpec((TB, 8), lambda i: (i, 0)),
        jax.ShapeDtypeStruct((NT, 8), f32))


def kernel(pos, z, batch, edge_index, params):
    N = pos.shape[0]
    E = edge_index.shape[1]
    MAXZ, H = params['emb'].shape
    G = params['blocks'][0]['mlp_w1'].shape[1]
    GP = 64

    NT = ((N + 1 + 2047) // 2048) * 2048
    NTS = ((N + 1 + 15) // 16) * 16
    CF, CB, CS = 56, 40, 112
    EPU = 1
    for u in (2 * NW * CF, 2 * NW * CB, 2 * NW * CS, 1024):
        EPU = EPU * u // math.gcd(EPU, u)
    EP = ((E + EPU - 1) // EPU) * EPU
    ECH = None
    for pt in (2, 4, 6, 8, 10, 12, 14, 16):
        if NT % (NW * pt) == 0 and NT // (NW * pt) <= 128 and (NT // (NW * pt)) % 8 == 0:
            ECH = NT // (NW * pt)
            break

    row = edge_index[0].astype(i32)
    col = edge_index[1].astype(i32)
    rowp = jnp.concatenate([row, jnp.full((EP - E,), N, i32)])
    colp = jnp.concatenate([col, jnp.full((EP - E,), N, i32)])
    rowF = rowp.reshape(EP // CF, CF)
    colF = colp.reshape(EP // CF, CF)
    rowB = rowp.reshape(EP // CB, CB)
    colB = colp.reshape(EP // CB, CB)
    rowS = rowp.reshape(EP // CS, CS)
    colS = colp.reshape(EP // CS, CS)
    pos8 = jnp.zeros((NT, 8), f32).at[:N, :3].set(pos.astype(f32))
    zp = jnp.zeros((NT,), i32).at[:N].set(z.astype(i32))
    z2 = zp.reshape(NT // ECH, ECH)
    batchp = jnp.full((NT, 1), jnp.int32(1 << 20)).at[:N, 0].set(batch.astype(i32))

    blocks = params['blocks']
    wT = []
    for blk in blocks:
        w1p = jnp.zeros((128, GP), f32).at[:, :G].set(blk['mlp_w1'])
        bias = jnp.zeros((8, 128), f32)
        bias = bias.at[0, :].set(blk['mlp_b1']).at[1, :].set(blk['mlp_b2'])
        bias = bias.at[2, :].set(blk['lin2_b']).at[3, :].set(blk['lin_b'])
        wT.append(dict(
            w1pT=w1p.T, w1p=w1p, w2T=blk['mlp_w2'].T, w2=blk['mlp_w2'],
            lin1T=blk['lin1_w'].T, lin1=blk['lin1_w'],
            lin2T=blk['lin2_w'].T, lin2=blk['lin2_w'],
            linT=blk['lin_w'].T, lin=blk['lin_w'], bias=bias))
    w1rT = params['w1'].T
    w1r = params['w1']
    small = jnp.zeros((8, 64), f32)
    small = small.at[0, :].set(params['b1']).at[1, :].set(params['w2'][0])
    small = small.at[2, :].set(jnp.broadcast_to(params['b2'], (64,)))

    gather_pos = _sc_gather((NT, 8), EP, CS)
    gather_emb = _sc_gather((MAXZ, H), NT, ECH)
    msg_fwd = _sc_msg_fwd(NT, NTS, EP, CF)
    msg_bwd = _sc_msg_bwd(NT, NTS, EP, CB)
    scat_force = _sc_scatter_force(NT, NTS, EP, CS)
    edge_geom = _edge_geom(EP)
    ew_fwd = _edge_w_fwd(EP, G, GP)
    ew_bwd = _edge_w_bwd(EP, G, GP)
    node_mm = _node_mm(NT)
    node_fwd = _node_fwd(NT)
    node_bwd1 = _node_bwd1(NT)
    node_bwd2 = _node_bwd2(NT)
    ro_fwd = _readout_fwd(NT)
    ro_bwd = _readout_bwd(NT)
    fvec = _force_vec(EP)
    fcomb = _force_combine(NT)

    prow = gather_pos(pos8, rowS)
    pcol = gather_pos(pos8, colS)
    d8, ew = edge_geom(prow, pcol)
    h = gather_emb(params['emb'], z2)

    hs, hxs, aggs, Ws = [], [], [], []
    for bi in range(len(blocks)):
        t = wT[bi]
        hs.append(h)
        hx = node_mm(h, t['lin1T'])
        hxs.append(hx)
        W = ew_fwd(ew, t['w1pT'], t['w2T'], t['bias'])
        Ws.append(W)
        aggpair = msg_fwd(hx, W, rowF, colF)
        h, agg = node_fwd(h, aggpair[0], aggpair[1], t['lin2T'], t['linT'],
                          t['bias'])
        aggs.append(agg)

    out8 = ro_fwd(h, batchp, w1rT, small)
    out = out8[0]

    dh = ro_bwd(h, w1rT, w1r, small)
    dew = jnp.zeros((EP, 1), f32)
    for bi in reversed(range(len(blocks))):
        t = wT[bi]
        dagg = node_bwd1(dh, aggs[bi], t['lin'], t['lin2T'], t['lin2'],
                         t['bias'])
        dW, dhxpair = msg_bwd(dagg, hxs[bi], Ws[bi], rowB, colB)
        dh = node_bwd2(dh, dhxpair[0], dhxpair[1], t['lin1'])
        dew = ew_bwd(ew, dW, dew, t['w1pT'], t['w1p'], t['w2T'], t['w2'],
                     t['bias'])

    vec = fvec(dew, ew, d8)
    fparts = scat_force(vec, rowS, colS)
    fneg = fcomb(fparts[0, 0], fparts[1, 0], fparts[0, 1], fparts[1, 1])
    forces = fneg[:N, :3]
    return out, forces


# speedup vs baseline: 2.3930x; 1.3848x over previous
import functools
import math

import jax
import jax.numpy as jnp
from jax import lax
from jax.experimental import pallas as pl
from jax.experimental.pallas import tpu as pltpu
from jax.experimental.pallas import tpu_sc as plsc

f32 = jnp.float32
i32 = jnp.int32

CUT = 5.0
NSC = 2
NTILE = 16
NW = NSC * NTILE
CH = 128
LANES = 16


def _ssp(x):
    return jnp.maximum(x, 0.0) + jnp.log(1.0 + jnp.exp(-jnp.abs(x))) - 0.6931471805599453


def _sig(x):
    return 1.0 / (1.0 + jnp.exp(-x))


def _mesh():
    return plsc.VectorSubcoreMesh(core_axis_name="c", subcore_axis_name="s")


def _wid():
    return lax.axis_index("c") * NTILE + lax.axis_index("s")


def _mul_inplace(dst, a, b, rows, cols):
    @pl.loop(0, rows)
    def _(r):
        for k in range(cols // LANES):
            sl = pl.ds(k * LANES, LANES)
            dst[r, sl] = a[r, sl] * b[r, sl]


def _zero_buf(buf, rows, cols):
    @pl.loop(0, rows)
    def _(r):
        for k in range(cols // LANES):
            buf[r, pl.ds(k * LANES, LANES)] = jnp.zeros((LANES,), f32)


def _zero_shared(zbuf, shared, rows_per_tile, chunk):
    s = lax.axis_index("s")
    nfull, rem = rows_per_tile // chunk, rows_per_tile % chunk
    for j in range(nfull):
        pltpu.sync_copy(zbuf, shared.at[pl.ds(s * rows_per_tile + j * chunk, chunk)])
    if rem:
        pltpu.sync_copy(zbuf.at[pl.ds(0, rem)],
                        shared.at[pl.ds(s * rows_per_tile + nfull * chunk, rem)])


def _stage_idx(idx2_hbm, idx2_v, w, per_tile):
    pltpu.sync_copy(idx2_hbm.at[pl.ds(w * per_tile, per_tile)], idx2_v)


def _sc_gather_all(NT, EP, CHP, ECH):
    pt_p = EP // NW // CHP
    pt_e = NT // NW // ECH
    assert pt_p % 2 == 0 and pt_e % 2 == 0

    @functools.partial(
        pl.kernel, mesh=_mesh(),
        out_type=(jax.ShapeDtypeStruct((NT, 128), f32),
                  jax.ShapeDtypeStruct((EP, 8), f32),
                  jax.ShapeDtypeStruct((EP, 8), f32)),
        compiler_params=pltpu.CompilerParams(use_tc_tiling_on_sc=False),
        scratch_types=[
            pltpu.VMEM((pt_p, CHP), i32),
            pltpu.VMEM((pt_p, CHP), i32),
            pltpu.VMEM((pt_e, ECH), i32),
            pltpu.VMEM((CHP, 8), f32),
            pltpu.VMEM((CHP, 8), f32),
            pltpu.VMEM((CHP, 8), f32),
            pltpu.VMEM((CHP, 8), f32),
            pltpu.VMEM((ECH, 128), f32),
            pltpu.VMEM((ECH, 128), f32),
            pltpu.SemaphoreType.DMA,
            pltpu.SemaphoreType.DMA,
            pltpu.SemaphoreType.DMA,
            pltpu.SemaphoreType.DMA,
            pltpu.SemaphoreType.DMA,
            pltpu.SemaphoreType.DMA,
        ],
    )
    def k(emb_hbm, pos_hbm, z2_hbm, row2_hbm, col2_hbm,
          h0_hbm, prow_hbm, pcol_hbm,
          ridx, cidx, zidx, rp0, rp1, rc0, rc1, e0, e1,
          sr0, sr1, sc0, sc1, se0, se1):
        w = _wid()
        rp = (rp0, rp1)
        rc = (rc0, rc1)
        ee = (e0, e1)
        sr = (sr0, sr1)
        scc = (sc0, sc1)
        se = (se0, se1)
        _stage_idx(z2_hbm, zidx, w, pt_e)
        _stage_idx(row2_hbm, ridx, w, pt_p)
        _stage_idx(col2_hbm, cidx, w, pt_p)

        def issue_e(j, b):
            pltpu.async_copy(emb_hbm.at[zidx.at[j]], ee[b], se[b])

        issue_e(0, 0)
        issue_e(1, 1)

        @pl.loop(0, pt_e, step=2)
        def _(g):
            for b in (0, 1):
                j = g + b
                base = (w * pt_e + j) * ECH
                pltpu.make_async_copy(emb_hbm.at[zidx.at[j]], ee[b], se[b]).wait()
                pltpu.sync_copy(ee[b], h0_hbm.at[pl.ds(base, ECH)])

                @pl.when(j + 2 < pt_e)
                def _():
                    issue_e(j + 2, b)

        def issue_p(j, b):
            pltpu.async_copy(pos_hbm.at[ridx.at[j]], rp[b], sr[b])
            pltpu.async_copy(pos_hbm.at[cidx.at[j]], rc[b], scc[b])

        issue_p(0, 0)
        issue_p(1, 1)

        @pl.loop(0, pt_p, step=2)
        def _(g):
            for b in (0, 1):
                j = g + b
                base = (w * pt_p + j) * CHP
                pltpu.make_async_copy(pos_hbm.at[ridx.at[j]], rp[b], sr[b]).wait()
                pltpu.make_async_copy(pos_hbm.at[cidx.at[j]], rc[b], scc[b]).wait()
                pltpu.sync_copy(rp[b], prow_hbm.at[pl.ds(base, CHP)])
                pltpu.sync_copy(rc[b], pcol_hbm.at[pl.ds(base, CHP)])

                @pl.when(j + 2 < pt_p)
                def _():
                    issue_p(j + 2, b)

    return k


def _sc_msg_fwd(NT, NTS, EP, CH, OFF, WD):
    per_tile = EP // NW // CH
    assert per_tile % 2 == 0

    @functools.partial(
        pl.kernel, mesh=_mesh(),
        out_type=jax.ShapeDtypeStruct((NSC, NT, 128), f32),
        compiler_params=pltpu.CompilerParams(use_tc_tiling_on_sc=False),
        scratch_types=[
            pltpu.VMEM((per_tile, CH), i32),
            pltpu.VMEM((per_tile, CH), i32),
            pltpu.VMEM((CH, 128), f32),
            pltpu.VMEM((CH, 128), f32),
            pltpu.VMEM((CH, 128), f32),
            pltpu.VMEM((CH, 128), f32),
            pltpu.VMEM_SHARED((NTS, 128), f32),
            pltpu.SemaphoreType.DMA,
            pltpu.SemaphoreType.DMA,
            pltpu.SemaphoreType.DMA,
            pltpu.SemaphoreType.DMA,
        ],
    )
    def k(hx_hbm, w_hbm, row2_hbm, col2_hbm, out_hbm, ridx, cidx,
          xj0, xj1, wv0, wv1, aggS, sg0, sg1, sw0, sw1):
        c = lax.axis_index("c")
        s = lax.axis_index("s")
        w = c * NTILE + s
        rpt = NTS // NTILE
        xj = (xj0, xj1)
        wv = (wv0, wv1)
        sg = (sg0, sg1)
        sw = (sw0, sw1)
        _zero_buf(wv0, CH, 128)
        _zero_shared(wv0, aggS, rpt, CH)
        _stage_idx(row2_hbm, ridx, w, per_tile)
        _stage_idx(col2_hbm, cidx, w, per_tile)
        plsc.subcore_barrier()

        def issue(j, b):
            base = (w * per_tile + j) * CH
            pltpu.async_copy(hx_hbm.at[ridx.at[j]], xj[b], sg[b])
            pltpu.async_copy(w_hbm.at[pl.ds(base, CH), pl.ds(OFF, 128)], wv[b], sw[b])

        issue(0, 0)
        issue(1, 1)

        @pl.loop(0, per_tile, step=2)
        def _(g):
            for b in (0, 1):
                j = g + b
                pltpu.make_async_copy(hx_hbm.at[ridx.at[j]], xj[b], sg[b]).wait()
                pltpu.make_async_copy(w_hbm.at[pl.ds(0, CH), pl.ds(OFF, 128)], wv[b], sw[b]).wait()
                _mul_inplace(wv[b], xj[b], wv[b], CH, 128)
                pltpu.sync_copy(wv[b], aggS.at[cidx.at[j]], add=True)

                @pl.when(j + 2 < per_tile)
                def _():
                    issue(j + 2, b)

        plsc.subcore_barrier()
        pltpu.sync_copy(aggS.at[pl.ds(s * rpt, rpt)],
                        out_hbm.at[c, pl.ds(s * rpt, rpt)])

    return k


def _sc_msg_bwd(NT, NTS, EP, CH, OFF, WD):
    per_tile = EP // NW // CH
    assert per_tile % 2 == 0

    @functools.partial(
        pl.kernel, mesh=_mesh(),
            out_type=(jax.ShapeDtypeStruct((EP, 128), f32),
                  jax.ShapeDtypeStruct((NSC, NT, 128), f32)),
        compiler_params=pltpu.CompilerParams(use_tc_tiling_on_sc=False),
        scratch_types=[
            pltpu.VMEM((per_tile, CH), i32),
            pltpu.VMEM((per_tile, CH), i32),
            pltpu.VMEM((CH, 128), f32),
            pltpu.VMEM((CH, 128), f32),
            pltpu.VMEM((CH, 128), f32),
            pltpu.VMEM((CH, 128), f32),
            pltpu.VMEM((CH, 128), f32),
            pltpu.VMEM((CH, 128), f32),
            pltpu.VMEM_SHARED((NTS, 128), f32),
            pltpu.SemaphoreType.DMA,
            pltpu.SemaphoreType.DMA,
            pltpu.SemaphoreType.DMA,
            pltpu.SemaphoreType.DMA,
            pltpu.SemaphoreType.DMA,
            pltpu.SemaphoreType.DMA,
        ],
    )
    def k(dagg_hbm, hx_hbm, w_hbm, row2_hbm, col2_hbm, dw_hbm, dhx_hbm,
          ridx, cidx, gv0, gv1, xj0, xj1, wv0, wv1, dhxS,
          sa0, sa1, sx0, sx1, sw0, sw1):
        c = lax.axis_index("c")
        s = lax.axis_index("s")
        w = c * NTILE + s
        rpt = NTS // NTILE
        gv = (gv0, gv1)
        xj = (xj0, xj1)
        wv = (wv0, wv1)
        sa = (sa0, sa1)
        sx = (sx0, sx1)
        sw = (sw0, sw1)
        _zero_buf(wv0, CH, 128)
        _zero_shared(wv0, dhxS, rpt, CH)
        _stage_idx(row2_hbm, ridx, w, per_tile)
        _stage_idx(col2_hbm, cidx, w, per_tile)
        plsc.subcore_barrier()

        def issue(j, b):
            base = (w * per_tile + j) * CH
            pltpu.async_copy(dagg_hbm.at[cidx.at[j]], gv[b], sa[b])
            pltpu.async_copy(hx_hbm.at[ridx.at[j]], xj[b], sx[b])
            pltpu.async_copy(w_hbm.at[pl.ds(base, CH), pl.ds(OFF, 128)], wv[b], sw[b])

        issue(0, 0)
        issue(1, 1)

        @pl.loop(0, per_tile, step=2)
        def _(g):
            for b in (0, 1):
                j = g + b
                base = (w * per_tile + j) * CH
                pltpu.make_async_copy(dagg_hbm.at[cidx.at[j]], gv[b], sa[b]).wait()
                pltpu.make_async_copy(hx_hbm.at[ridx.at[j]], xj[b], sx[b]).wait()
                pltpu.make_async_copy(w_hbm.at[pl.ds(0, CH), pl.ds(OFF, 128)], wv[b], sw[b]).wait()
                _mul_inplace(xj[b], gv[b], xj[b], CH, 128)
                _mul_inplace(wv[b], gv[b], wv[b], CH, 128)
                pltpu.sync_copy(xj[b], dw_hbm.at[pl.ds(base, CH)])
                pltpu.sync_copy(wv[b], dhxS.at[ridx.at[j]], add=True)

                @pl.when(j + 2 < per_tile)
                def _():
                    issue(j + 2, b)

        plsc.subcore_barrier()
        pltpu.sync_copy(dhxS.at[pl.ds(s * rpt, rpt)],
                        dhx_hbm.at[c, pl.ds(s * rpt, rpt)])

    return k


def _sc_msg_bwd_last(NT, EP, CH, OFF, WD):
    per_tile = EP // NW // CH
    assert per_tile % 2 == 0

    @functools.partial(
        pl.kernel, mesh=_mesh(),
        out_type=jax.ShapeDtypeStruct((EP, 128), f32),
        compiler_params=pltpu.CompilerParams(use_tc_tiling_on_sc=False),
        scratch_types=[
            pltpu.VMEM((per_tile, CH), i32),
            pltpu.VMEM((per_tile, CH), i32),
            pltpu.VMEM((CH, 128), f32),
            pltpu.VMEM((CH, 128), f32),
            pltpu.VMEM((CH, 128), f32),
            pltpu.VMEM((CH, 128), f32),
            pltpu.SemaphoreType.DMA,
            pltpu.SemaphoreType.DMA,
            pltpu.SemaphoreType.DMA,
            pltpu.SemaphoreType.DMA,
        ],
    )
    def k(dagg_hbm, hx_hbm, row2_hbm, col2_hbm, dw_hbm,
          ridx, cidx, gv0, gv1, xj0, xj1, sa0, sa1, sx0, sx1):
        w = _wid()
        gv = (gv0, gv1)
        xj = (xj0, xj1)
        sa = (sa0, sa1)
        sx = (sx0, sx1)
        _stage_idx(row2_hbm, ridx, w, per_tile)
        _stage_idx(col2_hbm, cidx, w, per_tile)

        def issue(j, b):
            pltpu.async_copy(dagg_hbm.at[cidx.at[j]], gv[b], sa[b])
            pltpu.async_copy(hx_hbm.at[ridx.at[j]], xj[b], sx[b])

        issue(0, 0)
        issue(1, 1)

        @pl.loop(0, per_tile, step=2)
        def _(g):
            for b in (0, 1):
                j = g + b
                base = (w * per_tile + j) * CH
                pltpu.make_async_copy(dagg_hbm.at[cidx.at[j]], gv[b], sa[b]).wait()
                pltpu.make_async_copy(hx_hbm.at[ridx.at[j]], xj[b], sx[b]).wait()
                _mul_inplace(xj[b], gv[b], xj[b], CH, 128)
                pltpu.sync_copy(xj[b], dw_hbm.at[pl.ds(base, CH)])

                @pl.when(j + 2 < per_tile)
                def _():
                    issue(j + 2, b)

    return k


def _sc_scatter_force(NT, NTS, EP, CH):
    per_tile = EP // NW // CH
    assert per_tile % 2 == 0

    @functools.partial(
        pl.kernel, mesh=_mesh(),
        out_type=jax.ShapeDtypeStruct((NSC, 2, NT, 8), f32),
        compiler_params=pltpu.CompilerParams(use_tc_tiling_on_sc=False),
        scratch_types=[
            pltpu.VMEM((per_tile, CH), i32),
            pltpu.VMEM((per_tile, CH), i32),
            pltpu.VMEM((CH, 8), f32),
            pltpu.VMEM((CH, 8), f32),
            pltpu.VMEM_SHARED((NTS, 8), f32),
            pltpu.VMEM_SHARED((NTS, 8), f32),
            pltpu.SemaphoreType.DMA,
            pltpu.SemaphoreType.DMA,
        ],
    )
    def k(vec_hbm, row2_hbm, col2_hbm, out_hbm, ridx, cidx, v0, v1,
          frS, fcS, sv0, sv1):
        c = lax.axis_index("c")
        s = lax.axis_index("s")
        w = c * NTILE + s
        rpt = NTS // NTILE
        vv = (v0, v1)
        sv = (sv0, sv1)
        _zero_buf(v0, CH, 8)
        _zero_shared(v0, frS, rpt, CH)
        _zero_shared(v0, fcS, rpt, CH)
        _stage_idx(row2_hbm, ridx, w, per_tile)
        _stage_idx(col2_hbm, cidx, w, per_tile)
        plsc.subcore_barrier()

        def issue(j, b):
            base = (w * per_tile + j) * CH
            pltpu.async_copy(vec_hbm.at[pl.ds(base, CH)], vv[b], sv[b])

        issue(0, 0)
        issue(1, 1)

        @pl.loop(0, per_tile, step=2)
        def _(g):
            for b in (0, 1):
                j = g + b
                pltpu.make_async_copy(vec_hbm.at[pl.ds(0, CH)], vv[b], sv[b]).wait()
                pltpu.sync_copy(vv[b], frS.at[ridx.at[j]], add=True)
                pltpu.sync_copy(vv[b], fcS.at[cidx.at[j]], add=True)

                @pl.when(j + 2 < per_tile)
                def _():
                    issue(j + 2, b)

        plsc.subcore_barrier()
        pltpu.sync_copy(frS.at[pl.ds(s * rpt, rpt)],
                        out_hbm.at[c, 0, pl.ds(s * rpt, rpt)])
        pltpu.sync_copy(fcS.at[pl.ds(s * rpt, rpt)],
                        out_hbm.at[c, 1, pl.ds(s * rpt, rpt)])

    return k


def _tc_call(body, grid, in_specs, out_specs, out_shape):
    return pl.pallas_call(
        body, grid=grid, in_specs=in_specs, out_specs=out_specs,
        out_shape=out_shape)


def _edge_geom(EP, TE=1024):
    def body(pr_ref, pc_ref, d_ref, ew_ref):
        d = pr_ref[...] - pc_ref[...]
        d_ref[...] = d
        ew_ref[...] = jnp.sqrt(jnp.sum(d * d, axis=1, keepdims=True) + 1e-12)

    return _tc_call(
        body, (EP // TE,),
        [pl.BlockSpec((TE, 8), lambda i: (i, 0))] * 2,
        [pl.BlockSpec((TE, 8), lambda i: (i, 0)),
         pl.BlockSpec((TE, 1), lambda i: (i, 0))],
        [jax.ShapeDtypeStruct((EP, 8), f32),
         jax.ShapeDtypeStruct((EP, 1), f32)])


def _smear(ew, G, GP):
    delta = CUT / (G - 1)
    coeff = -0.5 / delta ** 2
    off = lax.broadcasted_iota(i32, (ew.shape[0], GP), 1).astype(f32) * delta
    return jnp.exp(coeff * (ew - off) ** 2), off, coeff


def _edge_w_fwd_all(EP, G, GP, NB, TE=1024):
    def body(ew_ref, w1pT_ref, w2T_ref, bias_ref, wout_ref):
        ew = ew_ref[...]
        ea, _, _ = _smear(ew, G, GP)
        C = 0.5 * (jnp.cos(ew * (jnp.pi / CUT)) + 1.0)
        A1c = jnp.dot(ea, w1pT_ref[...], preferred_element_type=f32)
        for i in range(NB):
            sl = slice(i * 128, (i + 1) * 128)
            A1 = A1c[:, sl] + bias_ref[0:1, sl]
            S1 = _ssp(A1)
            W0 = jnp.dot(S1, w2T_ref[i * 128:(i + 1) * 128, :],
                         preferred_element_type=f32) + bias_ref[1:2, sl]
            wout_ref[:, sl] = W0 * C

    return _tc_call(
        body, (EP // TE,),
        [pl.BlockSpec((TE, 1), lambda i: (i, 0)),
         pl.BlockSpec((GP, NB * 128), lambda i: (0, 0)),
         pl.BlockSpec((NB * 128, 128), lambda i: (0, 0)),
         pl.BlockSpec((8, NB * 128), lambda i: (0, 0))],
        pl.BlockSpec((TE, NB * 128), lambda i: (i, 0)),
        jax.ShapeDtypeStruct((EP, NB * 128), f32))


def _edge_w_bwd_all(EP, G, GP, NB, TE=512):
    def body(ew_ref, dW0_ref, dW1_ref, dW2_ref, dW3_ref, d_ref,
             w1pT_ref, w1p_ref, w2T_ref, w2_ref, bias_ref, vec_ref):
        dWr = (dW0_ref, dW1_ref, dW2_ref, dW3_ref)
        ew = ew_ref[...]
        ea, off, coeff = _smear(ew, G, GP)
        C = 0.5 * (jnp.cos(ew * (jnp.pi / CUT)) + 1.0)
        dCdew = -0.5 * jnp.sin(ew * (jnp.pi / CUT)) * (jnp.pi / CUT)
        A1c = jnp.dot(ea, w1pT_ref[...], preferred_element_type=f32)
        dew = jnp.zeros((TE, 1), f32)
        deac = jnp.zeros((TE, GP), f32)
        for i in range(NB):
            sl = slice(i * 128, (i + 1) * 128)
            A1 = A1c[:, sl] + bias_ref[0:1, sl]
            S1 = _ssp(A1)
            W0 = jnp.dot(S1, w2T_ref[i * 128:(i + 1) * 128, :],
                         preferred_element_type=f32) + bias_ref[1:2, sl]
            dW = dWr[i][...]
            dC = jnp.sum(dW * W0, axis=1, keepdims=True)
            dW0 = dW * C
            dS1 = jnp.dot(dW0, w2_ref[i * 128:(i + 1) * 128, :],
                          preferred_element_type=f32)
            dA1 = dS1 * _sig(A1)
            deac = deac + jnp.dot(dA1, w1p_ref[:, i * GP:(i + 1) * GP],
                                  preferred_element_type=f32)
            dew = dew + dC * dCdew
        dea_dew = ea * (2.0 * coeff) * (ew - off)
        dew = dew + jnp.sum(deac * dea_dew, axis=1, keepdims=True)
        vec_ref[...] = (dew / ew) * d_ref[...]

    return _tc_call(
        body, (EP // TE,),
        [pl.BlockSpec((TE, 1), lambda i: (i, 0))] +
        [pl.BlockSpec((TE, 128), lambda i: (i, 0))] * NB +
        [pl.BlockSpec((TE, 8), lambda i: (i, 0)),
         pl.BlockSpec((GP, NB * 128), lambda i: (0, 0)),
         pl.BlockSpec((128, NB * GP), lambda i: (0, 0)),
         pl.BlockSpec((NB * 128, 128), lambda i: (0, 0)),
         pl.BlockSpec((NB * 128, 128), lambda i: (0, 0)),
         pl.BlockSpec((8, NB * 128), lambda i: (0, 0))],
        pl.BlockSpec((TE, 8), lambda i: (i, 0)),
        jax.ShapeDtypeStruct((EP, 8), f32))


def _node_mm(NT, TB=512):
    def body(x_ref, wT_ref, out_ref):
        out_ref[...] = jnp.dot(x_ref[...], wT_ref[...], preferred_element_type=f32)

    return _tc_call(
        body, (NT // TB,),
        [pl.BlockSpec((TB, 128), lambda i: (i, 0)),
         pl.BlockSpec((128, 128), lambda i: (0, 0))],
        pl.BlockSpec((TB, 128), lambda i: (i, 0)),
        jax.ShapeDtypeStruct((NT, 128), f32))


def _node_fwd(NT, TB=512):
    def body(h_ref, a0_ref, a1_ref, lin2T_ref, linT_ref, bias_ref,
             hout_ref, agg_ref):
        agg = a0_ref[...] + a1_ref[...]
        agg_ref[...] = agg
        A2 = jnp.dot(agg, lin2T_ref[...], preferred_element_type=f32) + bias_ref[2:3, :]
        S2 = _ssp(A2)
        hc = jnp.dot(S2, linT_ref[...], preferred_element_type=f32) + bias_ref[3:4, :]
        hout_ref[...] = h_ref[...] + hc

    return _tc_call(
        body, (NT // TB,),
        [pl.BlockSpec((TB, 128), lambda i: (i, 0))] * 3 +
        [pl.BlockSpec((128, 128), lambda i: (0, 0))] * 2 +
        [pl.BlockSpec((8, 128), lambda i: (0, 0))],
        [pl.BlockSpec((TB, 128), lambda i: (i, 0))] * 2,
        [jax.ShapeDtypeStruct((NT, 128), f32)] * 2)


def _node_bwd1(NT, TB=512):
    def body(dh_ref, agg_ref, lin_ref, lin2T_ref, lin2_ref, bias_ref, dagg_ref):
        dS2 = jnp.dot(dh_ref[...], lin_ref[...], preferred_element_type=f32)
        A2 = jnp.dot(agg_ref[...], lin2T_ref[...], preferred_element_type=f32) + bias_ref[2:3, :]
        dA2 = dS2 * _sig(A2)
        dagg_ref[...] = jnp.dot(dA2, lin2_ref[...], preferred_element_type=f32)

    return _tc_call(
        body, (NT // TB,),
        [pl.BlockSpec((TB, 128), lambda i: (i, 0))] * 2 +
        [pl.BlockSpec((128, 128), lambda i: (0, 0))] * 3 +
        [pl.BlockSpec((8, 128), lambda i: (0, 0))],
        pl.BlockSpec((TB, 128), lambda i: (i, 0)),
        jax.ShapeDtypeStruct((NT, 128), f32))


def _node_bwd2(NT, TB=512):
    def body(dh_ref, d0_ref, d1_ref, lin1_ref, out_ref):
        dhx = d0_ref[...] + d1_ref[...]
        out_ref[...] = dh_ref[...] + jnp.dot(dhx, lin1_ref[...],
                                             preferred_element_type=f32)

    return _tc_call(
        body, (NT // TB,),
        [pl.BlockSpec((TB, 128), lambda i: (i, 0))] * 3 +
        [pl.BlockSpec((128, 128), lambda i: (0, 0))],
        pl.BlockSpec((TB, 128), lambda i: (i, 0)),
        jax.ShapeDtypeStruct((NT, 128), f32))


def _readout_fwd(NT, TB=512):
    def body(h_ref, batch_ref, w1rT_ref, small_ref, out_ref):
        i = pl.program_id(0)
        A3 = jnp.dot(h_ref[...], w1rT_ref[...], preferred_element_type=f32) + small_ref[0:1, :]
        S3 = _ssp(A3)
        y = jnp.sum(S3 * small_ref[1:2, :], axis=1, keepdims=True) + small_ref[2:3, 0:1]
        b = batch_ref[...]
        y = jnp.where(b < 64, y, 0.0)
        onehot = (b == lax.broadcasted_iota(i32, (TB, 64), 1)).astype(f32)
        contrib = jnp.sum(y * onehot, axis=0, keepdims=True)

        @pl.when(i == 0)
        def _():
            out_ref[...] = jnp.zeros_like(out_ref)

        out_ref[0:1, :] = out_ref[0:1, :] + contrib

    return _tc_call(
        body, (NT // TB,),
        [pl.BlockSpec((TB, 128), lambda i: (i, 0)),
         pl.BlockSpec((TB, 1), lambda i: (i, 0)),
         pl.BlockSpec((128, 64), lambda i: (0, 0)),
         pl.BlockSpec((8, 64), lambda i: (0, 0))],
        pl.BlockSpec((8, 64), lambda i: (0, 0)),
        jax.ShapeDtypeStruct((8, 64), f32))


def _readout_bwd(NT, TB=512):
    def body(h_ref, w1rT_ref, w1r_ref, small_ref, dh_ref):
        A3 = jnp.dot(h_ref[...], w1rT_ref[...], preferred_element_type=f32) + small_ref[0:1, :]
        dA3 = _sig(A3) * small_ref[1:2, :]
        dh_ref[...] = jnp.dot(dA3, w1r_ref[...], preferred_element_type=f32)

    return _tc_call(
        body, (NT // TB,),
        [pl.BlockSpec((TB, 128), lambda i: (i, 0)),
         pl.BlockSpec((128, 64), lambda i: (0, 0)),
         pl.BlockSpec((64, 128), lambda i: (0, 0)),
         pl.BlockSpec((8, 64), lambda i: (0, 0))],
        pl.BlockSpec((TB, 128), lambda i: (i, 0)),
        jax.ShapeDtypeStruct((NT, 128), f32))


def _force_vec(EP, TE=1024):
    def body(dew_ref, ew_ref, d_ref, out_ref):
        out_ref[...] = (dew_ref[...] / ew_ref[...]) * d_ref[...]

    return _tc_call(
        body, (EP // TE,),
        [pl.BlockSpec((TE, 1), lambda i: (i, 0)),
         pl.BlockSpec((TE, 1), lambda i: (i, 0)),
         pl.BlockSpec((TE, 8), lambda i: (i, 0))],
        pl.BlockSpec((TE, 8), lambda i: (i, 0)),
        jax.ShapeDtypeStruct((EP, 8), f32))


def _force_combine(NT, TB=512):
    def body(r0_ref, r1_ref, c0_ref, c1_ref, out_ref):
        out_ref[...] = (c0_ref[...] + c1_ref[...]) - (r0_ref[...] + r1_ref[...])

    return _tc_call(
        body, (NT // TB,),
        [pl.BlockSpec((TB, 8), lambda i: (i, 0))] * 4,
        pl.BlockSpec((TB, 8), lambda i: (i, 0)),
        jax.ShapeDtypeStruct((NT, 8), f32))


def kernel(pos, z, batch, edge_index, params):
    N = pos.shape[0]
    E = edge_index.shape[1]
    MAXZ, H = params['emb'].shape
    G = params['blocks'][0]['mlp_w1'].shape[1]
    GP = 64
    NBK = len(params['blocks'])
    WD = NBK * 128

    NT = ((N + 1 + 2047) // 2048) * 2048
    NTS = ((N + 1 + 15) // 16) * 16
    CF, CB, CS = 56, 40, 112
    EPU = 1
    for u in (2 * NW * CF, 2 * NW * CB, 2 * NW * CS, 1024):
        EPU = EPU * u // math.gcd(EPU, u)
    EP = ((E + EPU - 1) // EPU) * EPU
    ECH = None
    for pt in (2, 4, 6, 8, 10, 12, 14, 16):
        if NT % (NW * pt) == 0 and NT // (NW * pt) <= 128 and (NT // (NW * pt)) % 8 == 0:
            ECH = NT // (NW * pt)
            break

    row = edge_index[0].astype(i32)
    col = edge_index[1].astype(i32)
    rowp = jnp.concatenate([row, jnp.full((EP - E,), N, i32)])
    colp = jnp.concatenate([col, jnp.full((EP - E,), N, i32)])
    rowF = rowp.reshape(EP // CF, CF)
    colF = colp.reshape(EP // CF, CF)
    rowB = rowp.reshape(EP // CB, CB)
    colB = colp.reshape(EP // CB, CB)
    rowS = rowp.reshape(EP // CS, CS)
    colS = colp.reshape(EP // CS, CS)
    pos8 = jnp.zeros((NT, 8), f32).at[:N, :3].set(pos.astype(f32))
    zp = jnp.zeros((NT,), i32).at[:N].set(z.astype(i32))
    z2 = zp.reshape(NT // ECH, ECH)
    batchp = jnp.full((NT, 1), jnp.int32(1 << 20)).at[:N, 0].set(batch.astype(i32))

    blocks = params['blocks']
    wT = []
    for blk in blocks:
        bias = jnp.zeros((8, 128), f32)
        bias = bias.at[2, :].set(blk['lin2_b']).at[3, :].set(blk['lin_b'])
        wT.append(dict(
            lin1T=blk['lin1_w'].T, lin1=blk['lin1_w'],
            lin2T=blk['lin2_w'].T, lin2=blk['lin2_w'],
            linT=blk['lin_w'].T, lin=blk['lin_w'], bias=bias))
    w1p_l = [jnp.zeros((128, GP), f32).at[:, :G].set(b['mlp_w1']) for b in blocks]
    w1pT_cat = jnp.concatenate([w.T for w in w1p_l], axis=1)
    w1p_cat = jnp.concatenate(w1p_l, axis=1)
    w2T_cat = jnp.concatenate([b['mlp_w2'].T for b in blocks], axis=0)
    w2_cat = jnp.concatenate([b['mlp_w2'] for b in blocks], axis=0)
    biasE = jnp.zeros((8, WD), f32)
    biasE = biasE.at[0, :].set(jnp.concatenate([b['mlp_b1'] for b in blocks]))
    biasE = biasE.at[1, :].set(jnp.concatenate([b['mlp_b2'] for b in blocks]))
    w1rT = params['w1'].T
    w1r = params['w1']
    small = jnp.zeros((8, 64), f32)
    small = small.at[0, :].set(params['b1']).at[1, :].set(params['w2'][0])
    small = small.at[2, :].set(jnp.broadcast_to(params['b2'], (64,)))

    gather_all = _sc_gather_all(NT, EP, CS, ECH)
    msg_fwd = [_sc_msg_fwd(NT, NTS, EP, CF, i * 128, WD) for i in range(NBK)]
    msg_bwd = [_sc_msg_bwd(NT, NTS, EP, CB, i * 128, WD) for i in range(NBK)]
    msg_bwd0 = _sc_msg_bwd_last(NT, EP, CS, 0, WD)
    scat_force = _sc_scatter_force(NT, NTS, EP, CS)
    edge_geom = _edge_geom(EP)
    ew_fwd = _edge_w_fwd_all(EP, G, GP, NBK)
    ew_bwd = _edge_w_bwd_all(EP, G, GP, NBK)
    node_mm = _node_mm(NT)
    node_fwd = _node_fwd(NT)
    node_bwd1 = _node_bwd1(NT)
    node_bwd2 = _node_bwd2(NT)
    ro_fwd = _readout_fwd(NT)
    ro_bwd = _readout_bwd(NT)
    fcomb = _force_combine(NT)

    h, prow, pcol = gather_all(params['emb'], pos8, z2, rowS, colS)
    d8, ew = edge_geom(prow, pcol)
    Wcat = ew_fwd(ew, w1pT_cat, w2T_cat, biasE)

    hxs, aggs = [], []
    for bi in range(NBK):
        t = wT[bi]
        hx = node_mm(h, t['lin1T'])
        hxs.append(hx)
        aggpair = msg_fwd[bi](hx, Wcat, rowF, colF)
        h, agg = node_fwd(h, aggpair[0], aggpair[1], t['lin2T'], t['linT'],
                          t['bias'])
        aggs.append(agg)

    out8 = ro_fwd(h, batchp, w1rT, small)
    out = out8[0]

    dh = ro_bwd(h, w1rT, w1r, small)
    dWs = [None] * NBK
    for bi in reversed(range(NBK)):
        t = wT[bi]
        dagg = node_bwd1(dh, aggs[bi], t['lin'], t['lin2T'], t['lin2'],
                         t['bias'])
        if bi > 0:
            dWs[bi], dhxpair = msg_bwd[bi](dagg, hxs[bi], Wcat, rowB, colB)
            dh = node_bwd2(dh, dhxpair[0], dhxpair[1], t['lin1'])
        else:
            dWs[bi] = msg_bwd0(dagg, hxs[bi], rowS, colS)

    vec = ew_bwd(ew, dWs[0], dWs[1], dWs[2], dWs[3], d8,
                 w1pT_cat, w1p_cat, w2T_cat, w2_cat, biasE)
    fparts = scat_force(vec, rowS, colS)
    fneg = fcomb(fparts[0, 0], fparts[1, 0], fparts[0, 1], fparts[1, 1])
    forces = fneg[:N, :3]
    return out, forces


# speedup vs baseline: 2.5339x; 1.0589x over previous
import functools
import math

import jax
import jax.numpy as jnp
from jax import lax
from jax.experimental import pallas as pl
from jax.experimental.pallas import tpu as pltpu
from jax.experimental.pallas import tpu_sc as plsc

f32 = jnp.float32
i32 = jnp.int32

CUT = 5.0
NSC = 2
NTILE = 16
NW = NSC * NTILE
CH = 128
LANES = 16


def _ssp(x):
    return jnp.maximum(x, 0.0) + jnp.log(1.0 + jnp.exp(-jnp.abs(x))) - 0.6931471805599453


def _sig(x):
    return 1.0 / (1.0 + jnp.exp(-x))


def _mesh():
    return plsc.VectorSubcoreMesh(core_axis_name="c", subcore_axis_name="s")


def _wid():
    return lax.axis_index("c") * NTILE + lax.axis_index("s")


def _mul_inplace(dst, a, b, rows, cols):
    @pl.loop(0, rows)
    def _(r):
        for k in range(cols // LANES):
            sl = pl.ds(k * LANES, LANES)
            dst[r, sl] = a[r, sl] * b[r, sl]


def _zero_buf(buf, rows, cols):
    @pl.loop(0, rows)
    def _(r):
        for k in range(cols // LANES):
            buf[r, pl.ds(k * LANES, LANES)] = jnp.zeros((LANES,), f32)


def _zero_shared(zbuf, shared, rows_per_tile, chunk):
    s = lax.axis_index("s")
    nfull, rem = rows_per_tile // chunk, rows_per_tile % chunk
    for j in range(nfull):
        pltpu.sync_copy(zbuf, shared.at[pl.ds(s * rows_per_tile + j * chunk, chunk)])
    if rem:
        pltpu.sync_copy(zbuf.at[pl.ds(0, rem)],
                        shared.at[pl.ds(s * rows_per_tile + nfull * chunk, rem)])


def _stage_idx(idx2_hbm, idx2_v, w, per_tile):
    pltpu.sync_copy(idx2_hbm.at[pl.ds(w * per_tile, per_tile)], idx2_v)


def _sc_gather_all(NT, EP, CHP, ECH):
    pt_p = EP // NW // CHP
    pt_e = NT // NW // ECH
    assert pt_p % 2 == 0 and pt_e % 2 == 0

    @functools.partial(
        pl.kernel, mesh=_mesh(),
        out_type=(jax.ShapeDtypeStruct((NT, 128), f32),
                  jax.ShapeDtypeStruct((EP, 8), f32),
                  jax.ShapeDtypeStruct((EP, 8), f32)),
        compiler_params=pltpu.CompilerParams(use_tc_tiling_on_sc=False),
        scratch_types=[
            pltpu.VMEM((pt_p, CHP), i32),
            pltpu.VMEM((pt_p, CHP), i32),
            pltpu.VMEM((pt_e, ECH), i32),
            pltpu.VMEM((CHP, 8), f32),
            pltpu.VMEM((CHP, 8), f32),
            pltpu.VMEM((CHP, 8), f32),
            pltpu.VMEM((CHP, 8), f32),
            pltpu.VMEM((ECH, 128), f32),
            pltpu.VMEM((ECH, 128), f32),
            pltpu.SemaphoreType.DMA,
            pltpu.SemaphoreType.DMA,
            pltpu.SemaphoreType.DMA,
            pltpu.SemaphoreType.DMA,
            pltpu.SemaphoreType.DMA,
            pltpu.SemaphoreType.DMA,
        ],
    )
    def k(emb_hbm, pos_hbm, z2_hbm, row2_hbm, col2_hbm,
          h0_hbm, prow_hbm, pcol_hbm,
          ridx, cidx, zidx, rp0, rp1, rc0, rc1, e0, e1,
          sr0, sr1, sc0, sc1, se0, se1):
        w = _wid()
        rp = (rp0, rp1)
        rc = (rc0, rc1)
        ee = (e0, e1)
        sr = (sr0, sr1)
        scc = (sc0, sc1)
        se = (se0, se1)
        _stage_idx(z2_hbm, zidx, w, pt_e)
        _stage_idx(row2_hbm, ridx, w, pt_p)
        _stage_idx(col2_hbm, cidx, w, pt_p)

        def issue_e(j, b):
            pltpu.async_copy(emb_hbm.at[zidx.at[j]], ee[b], se[b])

        issue_e(0, 0)
        issue_e(1, 1)

        @pl.loop(0, pt_e, step=2)
        def _(g):
            for b in (0, 1):
                j = g + b
                base = (w * pt_e + j) * ECH
                pltpu.make_async_copy(emb_hbm.at[zidx.at[j]], ee[b], se[b]).wait()
                pltpu.sync_copy(ee[b], h0_hbm.at[pl.ds(base, ECH)])

                @pl.when(j + 2 < pt_e)
                def _():
                    issue_e(j + 2, b)

        def issue_p(j, b):
            pltpu.async_copy(pos_hbm.at[ridx.at[j]], rp[b], sr[b])
            pltpu.async_copy(pos_hbm.at[cidx.at[j]], rc[b], scc[b])

        issue_p(0, 0)
        issue_p(1, 1)

        @pl.loop(0, pt_p, step=2)
        def _(g):
            for b in (0, 1):
                j = g + b
                base = (w * pt_p + j) * CHP
                pltpu.make_async_copy(pos_hbm.at[ridx.at[j]], rp[b], sr[b]).wait()
                pltpu.make_async_copy(pos_hbm.at[cidx.at[j]], rc[b], scc[b]).wait()
                pltpu.sync_copy(rp[b], prow_hbm.at[pl.ds(base, CHP)])
                pltpu.sync_copy(rc[b], pcol_hbm.at[pl.ds(base, CHP)])

                @pl.when(j + 2 < pt_p)
                def _():
                    issue_p(j + 2, b)

    return k


def _sc_msg_fwd(NT, NTS, EP, CH):
    per_tile = EP // NW // CH
    assert per_tile % 2 == 0

    @functools.partial(
        pl.kernel, mesh=_mesh(),
        out_type=jax.ShapeDtypeStruct((NSC, NT, 128), f32),
        compiler_params=pltpu.CompilerParams(use_tc_tiling_on_sc=False),
        scratch_types=[
            pltpu.VMEM((per_tile, CH), i32),
            pltpu.VMEM((per_tile, CH), i32),
            pltpu.VMEM((CH, 128), f32),
            pltpu.VMEM((CH, 128), f32),
            pltpu.VMEM((CH, 128), f32),
            pltpu.VMEM((CH, 128), f32),
            pltpu.VMEM_SHARED((NTS, 128), f32),
            pltpu.SemaphoreType.DMA,
            pltpu.SemaphoreType.DMA,
            pltpu.SemaphoreType.DMA,
            pltpu.SemaphoreType.DMA,
        ],
    )
    def k(hx_hbm, w_hbm, row2_hbm, col2_hbm, out_hbm, ridx, cidx,
          xj0, xj1, wv0, wv1, aggS, sg0, sg1, sw0, sw1):
        c = lax.axis_index("c")
        s = lax.axis_index("s")
        w = c * NTILE + s
        rpt = NTS // NTILE
        xj = (xj0, xj1)
        wv = (wv0, wv1)
        sg = (sg0, sg1)
        sw = (sw0, sw1)
        _zero_buf(wv0, CH, 128)
        _zero_shared(wv0, aggS, rpt, CH)
        _stage_idx(row2_hbm, ridx, w, per_tile)
        _stage_idx(col2_hbm, cidx, w, per_tile)
        plsc.subcore_barrier()

        def issue(j, b):
            base = (w * per_tile + j) * CH
            pltpu.async_copy(hx_hbm.at[ridx.at[j]], xj[b], sg[b])
            pltpu.async_copy(w_hbm.at[pl.ds(base, CH)], wv[b], sw[b])

        issue(0, 0)
        issue(1, 1)

        @pl.loop(0, per_tile, step=2)
        def _(g):
            for b in (0, 1):
                j = g + b
                pltpu.make_async_copy(hx_hbm.at[ridx.at[j]], xj[b], sg[b]).wait()
                pltpu.make_async_copy(w_hbm.at[pl.ds(0, CH)], wv[b], sw[b]).wait()
                _mul_inplace(wv[b], xj[b], wv[b], CH, 128)
                pltpu.sync_copy(wv[b], aggS.at[cidx.at[j]], add=True)

                @pl.when(j + 2 < per_tile)
                def _():
                    issue(j + 2, b)

        plsc.subcore_barrier()
        pltpu.sync_copy(aggS.at[pl.ds(s * rpt, rpt)],
                        out_hbm.at[c, pl.ds(s * rpt, rpt)])

    return k


def _sc_msg_bwd(NT, NTS, EP, CH):
    per_tile = EP // NW // CH
    assert per_tile % 2 == 0

    @functools.partial(
        pl.kernel, mesh=_mesh(),
            out_type=(jax.ShapeDtypeStruct((EP, 128), f32),
                  jax.ShapeDtypeStruct((NSC, NT, 128), f32)),
        compiler_params=pltpu.CompilerParams(use_tc_tiling_on_sc=False),
        scratch_types=[
            pltpu.VMEM((per_tile, CH), i32),
            pltpu.VMEM((per_tile, CH), i32),
            pltpu.VMEM((CH, 128), f32),
            pltpu.VMEM((CH, 128), f32),
            pltpu.VMEM((CH, 128), f32),
            pltpu.VMEM((CH, 128), f32),
            pltpu.VMEM((CH, 128), f32),
            pltpu.VMEM((CH, 128), f32),
            pltpu.VMEM_SHARED((NTS, 128), f32),
            pltpu.SemaphoreType.DMA,
            pltpu.SemaphoreType.DMA,
            pltpu.SemaphoreType.DMA,
            pltpu.SemaphoreType.DMA,
            pltpu.SemaphoreType.DMA,
            pltpu.SemaphoreType.DMA,
        ],
    )
    def k(dagg_hbm, hx_hbm, w_hbm, row2_hbm, col2_hbm, dw_hbm, dhx_hbm,
          ridx, cidx, gv0, gv1, xj0, xj1, wv0, wv1, dhxS,
          sa0, sa1, sx0, sx1, sw0, sw1):
        c = lax.axis_index("c")
        s = lax.axis_index("s")
        w = c * NTILE + s
        rpt = NTS // NTILE
        gv = (gv0, gv1)
        xj = (xj0, xj1)
        wv = (wv0, wv1)
        sa = (sa0, sa1)
        sx = (sx0, sx1)
        sw = (sw0, sw1)
        _zero_buf(wv0, CH, 128)
        _zero_shared(wv0, dhxS, rpt, CH)
        _stage_idx(row2_hbm, ridx, w, per_tile)
        _stage_idx(col2_hbm, cidx, w, per_tile)
        plsc.subcore_barrier()

        def issue(j, b):
            base = (w * per_tile + j) * CH
            pltpu.async_copy(dagg_hbm.at[cidx.at[j]], gv[b], sa[b])
            pltpu.async_copy(hx_hbm.at[ridx.at[j]], xj[b], sx[b])
            pltpu.async_copy(w_hbm.at[pl.ds(base, CH)], wv[b], sw[b])

        issue(0, 0)
        issue(1, 1)

        @pl.loop(0, per_tile, step=2)
        def _(g):
            for b in (0, 1):
                j = g + b
                base = (w * per_tile + j) * CH
                pltpu.make_async_copy(dagg_hbm.at[cidx.at[j]], gv[b], sa[b]).wait()
                pltpu.make_async_copy(hx_hbm.at[ridx.at[j]], xj[b], sx[b]).wait()
                pltpu.make_async_copy(w_hbm.at[pl.ds(0, CH)], wv[b], sw[b]).wait()
                _mul_inplace(xj[b], gv[b], xj[b], CH, 128)
                _mul_inplace(wv[b], gv[b], wv[b], CH, 128)
                pltpu.sync_copy(xj[b], dw_hbm.at[pl.ds(base, CH)])
                pltpu.sync_copy(wv[b], dhxS.at[ridx.at[j]], add=True)

                @pl.when(j + 2 < per_tile)
                def _():
                    issue(j + 2, b)

        plsc.subcore_barrier()
        pltpu.sync_copy(dhxS.at[pl.ds(s * rpt, rpt)],
                        dhx_hbm.at[c, pl.ds(s * rpt, rpt)])

    return k


def _sc_msg_bwd_last(NT, EP, CH):
    per_tile = EP // NW // CH
    assert per_tile % 2 == 0

    @functools.partial(
        pl.kernel, mesh=_mesh(),
        out_type=jax.ShapeDtypeStruct((EP, 128), f32),
        compiler_params=pltpu.CompilerParams(use_tc_tiling_on_sc=False),
        scratch_types=[
            pltpu.VMEM((per_tile, CH), i32),
            pltpu.VMEM((per_tile, CH), i32),
            pltpu.VMEM((CH, 128), f32),
            pltpu.VMEM((CH, 128), f32),
            pltpu.VMEM((CH, 128), f32),
            pltpu.VMEM((CH, 128), f32),
            pltpu.SemaphoreType.DMA,
            pltpu.SemaphoreType.DMA,
            pltpu.SemaphoreType.DMA,
            pltpu.SemaphoreType.DMA,
        ],
    )
    def k(dagg_hbm, hx_hbm, row2_hbm, col2_hbm, dw_hbm,
          ridx, cidx, gv0, gv1, xj0, xj1, sa0, sa1, sx0, sx1):
        w = _wid()
        gv = (gv0, gv1)
        xj = (xj0, xj1)
        sa = (sa0, sa1)
        sx = (sx0, sx1)
        _stage_idx(row2_hbm, ridx, w, per_tile)
        _stage_idx(col2_hbm, cidx, w, per_tile)

        def issue(j, b):
            pltpu.async_copy(dagg_hbm.at[cidx.at[j]], gv[b], sa[b])
            pltpu.async_copy(hx_hbm.at[ridx.at[j]], xj[b], sx[b])

        issue(0, 0)
        issue(1, 1)

        @pl.loop(0, per_tile, step=2)
        def _(g):
            for b in (0, 1):
                j = g + b
                base = (w * per_tile + j) * CH
                pltpu.make_async_copy(dagg_hbm.at[cidx.at[j]], gv[b], sa[b]).wait()
                pltpu.make_async_copy(hx_hbm.at[ridx.at[j]], xj[b], sx[b]).wait()
                _mul_inplace(xj[b], gv[b], xj[b], CH, 128)
                pltpu.sync_copy(xj[b], dw_hbm.at[pl.ds(base, CH)])

                @pl.when(j + 2 < per_tile)
                def _():
                    issue(j + 2, b)

    return k


def _sc_scatter_force(NT, NTS, EP, CH):
    per_tile = EP // NW // CH
    assert per_tile % 2 == 0

    @functools.partial(
        pl.kernel, mesh=_mesh(),
        out_type=jax.ShapeDtypeStruct((NSC, 2, NT, 8), f32),
        compiler_params=pltpu.CompilerParams(use_tc_tiling_on_sc=False),
        scratch_types=[
            pltpu.VMEM((per_tile, CH), i32),
            pltpu.VMEM((per_tile, CH), i32),
            pltpu.VMEM((CH, 8), f32),
            pltpu.VMEM((CH, 8), f32),
            pltpu.VMEM_SHARED((NTS, 8), f32),
            pltpu.VMEM_SHARED((NTS, 8), f32),
            pltpu.SemaphoreType.DMA,
            pltpu.SemaphoreType.DMA,
        ],
    )
    def k(vec_hbm, row2_hbm, col2_hbm, out_hbm, ridx, cidx, v0, v1,
          frS, fcS, sv0, sv1):
        c = lax.axis_index("c")
        s = lax.axis_index("s")
        w = c * NTILE + s
        rpt = NTS // NTILE
        vv = (v0, v1)
        sv = (sv0, sv1)
        _zero_buf(v0, CH, 8)
        _zero_shared(v0, frS, rpt, CH)
        _zero_shared(v0, fcS, rpt, CH)
        _stage_idx(row2_hbm, ridx, w, per_tile)
        _stage_idx(col2_hbm, cidx, w, per_tile)
        plsc.subcore_barrier()

        def issue(j, b):
            base = (w * per_tile + j) * CH
            pltpu.async_copy(vec_hbm.at[pl.ds(base, CH)], vv[b], sv[b])

        issue(0, 0)
        issue(1, 1)

        @pl.loop(0, per_tile, step=2)
        def _(g):
            for b in (0, 1):
                j = g + b
                pltpu.make_async_copy(vec_hbm.at[pl.ds(0, CH)], vv[b], sv[b]).wait()
                pltpu.sync_copy(vv[b], frS.at[ridx.at[j]], add=True)
                pltpu.sync_copy(vv[b], fcS.at[cidx.at[j]], add=True)

                @pl.when(j + 2 < per_tile)
                def _():
                    issue(j + 2, b)

        plsc.subcore_barrier()
        pltpu.sync_copy(frS.at[pl.ds(s * rpt, rpt)],
                        out_hbm.at[c, 0, pl.ds(s * rpt, rpt)])
        pltpu.sync_copy(fcS.at[pl.ds(s * rpt, rpt)],
                        out_hbm.at[c, 1, pl.ds(s * rpt, rpt)])

    return k


def _tc_call(body, grid, in_specs, out_specs, out_shape):
    return pl.pallas_call(
        body, grid=grid, in_specs=in_specs, out_specs=out_specs,
        out_shape=out_shape)


def _edge_geom(EP, TE=1024):
    def body(pr_ref, pc_ref, d_ref, ew_ref):
        d = pr_ref[...] - pc_ref[...]
        d_ref[...] = d
        ew_ref[...] = jnp.sqrt(jnp.sum(d * d, axis=1, keepdims=True) + 1e-12)

    return _tc_call(
        body, (EP // TE,),
        [pl.BlockSpec((TE, 8), lambda i: (i, 0))] * 2,
        [pl.BlockSpec((TE, 8), lambda i: (i, 0)),
         pl.BlockSpec((TE, 1), lambda i: (i, 0))],
        [jax.ShapeDtypeStruct((EP, 8), f32),
         jax.ShapeDtypeStruct((EP, 1), f32)])


def _smear(ew, G, GP):
    delta = CUT / (G - 1)
    coeff = -0.5 / delta ** 2
    off = lax.broadcasted_iota(i32, (ew.shape[0], GP), 1).astype(f32) * delta
    return jnp.exp(coeff * (ew - off) ** 2), off, coeff


def _edge_w_fwd_all(EP, G, GP, NB, TE=1024):
    def body(ew_ref, w1pT_ref, w2T_ref, bias_ref, *wout_refs):
        ew = ew_ref[...]
        ea, _, _ = _smear(ew, G, GP)
        C = 0.5 * (jnp.cos(ew * (jnp.pi / CUT)) + 1.0)
        A1c = jnp.dot(ea, w1pT_ref[...], preferred_element_type=f32)
        for i in range(NB):
            sl = slice(i * 128, (i + 1) * 128)
            A1 = A1c[:, sl] + bias_ref[0:1, sl]
            S1 = _ssp(A1)
            W0 = jnp.dot(S1, w2T_ref[i * 128:(i + 1) * 128, :],
                         preferred_element_type=f32) + bias_ref[1:2, sl]
            wout_refs[i][...] = W0 * C

    return _tc_call(
        body, (EP // TE,),
        [pl.BlockSpec((TE, 1), lambda i: (i, 0)),
         pl.BlockSpec((GP, NB * 128), lambda i: (0, 0)),
         pl.BlockSpec((NB * 128, 128), lambda i: (0, 0)),
         pl.BlockSpec((8, NB * 128), lambda i: (0, 0))],
        [pl.BlockSpec((TE, 128), lambda i: (i, 0))] * NB,
        [jax.ShapeDtypeStruct((EP, 128), f32)] * NB)


def _edge_w_bwd_all(EP, G, GP, NB, TE=512):
    def body(ew_ref, dW0_ref, dW1_ref, dW2_ref, dW3_ref, d_ref,
             w1pT_ref, w1p_ref, w2T_ref, w2_ref, bias_ref, vec_ref):
        dWr = (dW0_ref, dW1_ref, dW2_ref, dW3_ref)
        ew = ew_ref[...]
        ea, off, coeff = _smear(ew, G, GP)
        C = 0.5 * (jnp.cos(ew * (jnp.pi / CUT)) + 1.0)
        dCdew = -0.5 * jnp.sin(ew * (jnp.pi / CUT)) * (jnp.pi / CUT)
        A1c = jnp.dot(ea, w1pT_ref[...], preferred_element_type=f32)
        dew = jnp.zeros((TE, 1), f32)
        deac = jnp.zeros((TE, GP), f32)
        for i in range(NB):
            sl = slice(i * 128, (i + 1) * 128)
            A1 = A1c[:, sl] + bias_ref[0:1, sl]
            S1 = _ssp(A1)
            W0 = jnp.dot(S1, w2T_ref[i * 128:(i + 1) * 128, :],
                         preferred_element_type=f32) + bias_ref[1:2, sl]
            dW = dWr[i][...]
            dC = jnp.sum(dW * W0, axis=1, keepdims=True)
            dW0 = dW * C
            dS1 = jnp.dot(dW0, w2_ref[i * 128:(i + 1) * 128, :],
                          preferred_element_type=f32)
            dA1 = dS1 * _sig(A1)
            deac = deac + jnp.dot(dA1, w1p_ref[:, i * GP:(i + 1) * GP],
                                  preferred_element_type=f32)
            dew = dew + dC * dCdew
        dea_dew = ea * (2.0 * coeff) * (ew - off)
        dew = dew + jnp.sum(deac * dea_dew, axis=1, keepdims=True)
        vec_ref[...] = (dew / ew) * d_ref[...]

    return _tc_call(
        body, (EP // TE,),
        [pl.BlockSpec((TE, 1), lambda i: (i, 0))] +
        [pl.BlockSpec((TE, 128), lambda i: (i, 0))] * NB +
        [pl.BlockSpec((TE, 8), lambda i: (i, 0)),
         pl.BlockSpec((GP, NB * 128), lambda i: (0, 0)),
         pl.BlockSpec((128, NB * GP), lambda i: (0, 0)),
         pl.BlockSpec((NB * 128, 128), lambda i: (0, 0)),
         pl.BlockSpec((NB * 128, 128), lambda i: (0, 0)),
         pl.BlockSpec((8, NB * 128), lambda i: (0, 0))],
        pl.BlockSpec((TE, 8), lambda i: (i, 0)),
        jax.ShapeDtypeStruct((EP, 8), f32))


def _node_mm(NT, TB=512):
    def body(x_ref, wT_ref, out_ref):
        out_ref[...] = jnp.dot(x_ref[...], wT_ref[...], preferred_element_type=f32)

    return _tc_call(
        body, (NT // TB,),
        [pl.BlockSpec((TB, 128), lambda i: (i, 0)),
         pl.BlockSpec((128, 128), lambda i: (0, 0))],
        pl.BlockSpec((TB, 128), lambda i: (i, 0)),
        jax.ShapeDtypeStruct((NT, 128), f32))


def _node_fwd(NT, TB=512):
    def body(h_ref, a0_ref, a1_ref, lin2T_ref, linT_ref, bias_ref,
             hout_ref, agg_ref):
        agg = a0_ref[...] + a1_ref[...]
        agg_ref[...] = agg
        A2 = jnp.dot(agg, lin2T_ref[...], preferred_element_type=f32) + bias_ref[2:3, :]
        S2 = _ssp(A2)
        hc = jnp.dot(S2, linT_ref[...], preferred_element_type=f32) + bias_ref[3:4, :]
        hout_ref[...] = h_ref[...] + hc

    return _tc_call(
        body, (NT // TB,),
        [pl.BlockSpec((TB, 128), lambda i: (i, 0))] * 3 +
        [pl.BlockSpec((128, 128), lambda i: (0, 0))] * 2 +
        [pl.BlockSpec((8, 128), lambda i: (0, 0))],
        [pl.BlockSpec((TB, 128), lambda i: (i, 0))] * 2,
        [jax.ShapeDtypeStruct((NT, 128), f32)] * 2)


def _node_bwd1(NT, TB=512):
    def body(dh_ref, agg_ref, lin_ref, lin2T_ref, lin2_ref, bias_ref, dagg_ref):
        dS2 = jnp.dot(dh_ref[...], lin_ref[...], preferred_element_type=f32)
        A2 = jnp.dot(agg_ref[...], lin2T_ref[...], preferred_element_type=f32) + bias_ref[2:3, :]
        dA2 = dS2 * _sig(A2)
        dagg_ref[...] = jnp.dot(dA2, lin2_ref[...], preferred_element_type=f32)

    return _tc_call(
        body, (NT // TB,),
        [pl.BlockSpec((TB, 128), lambda i: (i, 0))] * 2 +
        [pl.BlockSpec((128, 128), lambda i: (0, 0))] * 3 +
        [pl.BlockSpec((8, 128), lambda i: (0, 0))],
        pl.BlockSpec((TB, 128), lambda i: (i, 0)),
        jax.ShapeDtypeStruct((NT, 128), f32))


def _node_bwd2(NT, TB=512):
    def body(dh_ref, d0_ref, d1_ref, lin1_ref, out_ref):
        dhx = d0_ref[...] + d1_ref[...]
        out_ref[...] = dh_ref[...] + jnp.dot(dhx, lin1_ref[...],
                                             preferred_element_type=f32)

    return _tc_call(
        body, (NT // TB,),
        [pl.BlockSpec((TB, 128), lambda i: (i, 0))] * 3 +
        [pl.BlockSpec((128, 128), lambda i: (0, 0))],
        pl.BlockSpec((TB, 128), lambda i: (i, 0)),
        jax.ShapeDtypeStruct((NT, 128), f32))


def _readout_fwd(NT, TB=512):
    def body(h_ref, batch_ref, w1rT_ref, small_ref, out_ref):
        i = pl.program_id(0)
        A3 = jnp.dot(h_ref[...], w1rT_ref[...], preferred_element_type=f32) + small_ref[0:1, :]
        S3 = _ssp(A3)
        y = jnp.sum(S3 * small_ref[1:2, :], axis=1, keepdims=True) + small_ref[2:3, 0:1]
        b = batch_ref[...]
        y = jnp.where(b < 64, y, 0.0)
        onehot = (b == lax.broadcasted_iota(i32, (TB, 64), 1)).astype(f32)
        contrib = jnp.sum(y * onehot, axis=0, keepdims=True)

        @pl.when(i == 0)
        def _():
            out_ref[...] = jnp.zeros_like(out_ref)

        out_ref[0:1, :] = out_ref[0:1, :] + contrib

    return _tc_call(
        body, (NT // TB,),
        [pl.BlockSpec((TB, 128), lambda i: (i, 0)),
         pl.BlockSpec((TB, 1), lambda i: (i, 0)),
         pl.BlockSpec((128, 64), lambda i: (0, 0)),
         pl.BlockSpec((8, 64), lambda i: (0, 0))],
        pl.BlockSpec((8, 64), lambda i: (0, 0)),
        jax.ShapeDtypeStruct((8, 64), f32))


def _readout_bwd(NT, TB=512):
    def body(h_ref, w1rT_ref, w1r_ref, small_ref, dh_ref):
        A3 = jnp.dot(h_ref[...], w1rT_ref[...], preferred_element_type=f32) + small_ref[0:1, :]
        dA3 = _sig(A3) * small_ref[1:2, :]
        dh_ref[...] = jnp.dot(dA3, w1r_ref[...], preferred_element_type=f32)

    return _tc_call(
        body, (NT // TB,),
        [pl.BlockSpec((TB, 128), lambda i: (i, 0)),
         pl.BlockSpec((128, 64), lambda i: (0, 0)),
         pl.BlockSpec((64, 128), lambda i: (0, 0)),
         pl.BlockSpec((8, 64), lambda i: (0, 0))],
        pl.BlockSpec((TB, 128), lambda i: (i, 0)),
        jax.ShapeDtypeStruct((NT, 128), f32))


def _force_vec(EP, TE=1024):
    def body(dew_ref, ew_ref, d_ref, out_ref):
        out_ref[...] = (dew_ref[...] / ew_ref[...]) * d_ref[...]

    return _tc_call(
        body, (EP // TE,),
        [pl.BlockSpec((TE, 1), lambda i: (i, 0)),
         pl.BlockSpec((TE, 1), lambda i: (i, 0)),
         pl.BlockSpec((TE, 8), lambda i: (i, 0))],
        pl.BlockSpec((TE, 8), lambda i: (i, 0)),
        jax.ShapeDtypeStruct((EP, 8), f32))


def _force_combine(NT, TB=512):
    def body(r0_ref, r1_ref, c0_ref, c1_ref, out_ref):
        out_ref[...] = (c0_ref[...] + c1_ref[...]) - (r0_ref[...] + r1_ref[...])

    return _tc_call(
        body, (NT // TB,),
        [pl.BlockSpec((TB, 8), lambda i: (i, 0))] * 4,
        pl.BlockSpec((TB, 8), lambda i: (i, 0)),
        jax.ShapeDtypeStruct((NT, 8), f32))


def kernel(pos, z, batch, edge_index, params):
    N = pos.shape[0]
    E = edge_index.shape[1]
    MAXZ, H = params['emb'].shape
    G = params['blocks'][0]['mlp_w1'].shape[1]
    GP = 64
    NBK = len(params['blocks'])
    WD = NBK * 128

    NT = ((N + 1 + 2047) // 2048) * 2048
    NTS = ((N + 1 + 15) // 16) * 16
    CF, CB, CS = 56, 40, 112
    EPU = 1
    for u in (2 * NW * CF, 2 * NW * CB, 2 * NW * CS, 1024):
        EPU = EPU * u // math.gcd(EPU, u)
    EP = ((E + EPU - 1) // EPU) * EPU
    ECH = None
    for pt in (2, 4, 6, 8, 10, 12, 14, 16):
        if NT % (NW * pt) == 0 and NT // (NW * pt) <= 128 and (NT // (NW * pt)) % 8 == 0:
            ECH = NT // (NW * pt)
            break

    row = edge_index[0].astype(i32)
    col = edge_index[1].astype(i32)
    rowp = jnp.concatenate([row, jnp.full((EP - E,), N, i32)])
    colp = jnp.concatenate([col, jnp.full((EP - E,), N, i32)])
    rowF = rowp.reshape(EP // CF, CF)
    colF = colp.reshape(EP // CF, CF)
    rowB = rowp.reshape(EP // CB, CB)
    colB = colp.reshape(EP // CB, CB)
    rowS = rowp.reshape(EP // CS, CS)
    colS = colp.reshape(EP // CS, CS)
    pos8 = jnp.zeros((NT, 8), f32).at[:N, :3].set(pos.astype(f32))
    zp = jnp.zeros((NT,), i32).at[:N].set(z.astype(i32))
    z2 = zp.reshape(NT // ECH, ECH)
    batchp = jnp.full((NT, 1), jnp.int32(1 << 20)).at[:N, 0].set(batch.astype(i32))

    blocks = params['blocks']
    wT = []
    for blk in blocks:
        bias = jnp.zeros((8, 128), f32)
        bias = bias.at[2, :].set(blk['lin2_b']).at[3, :].set(blk['lin_b'])
        wT.append(dict(
            lin1T=blk['lin1_w'].T, lin1=blk['lin1_w'],
            lin2T=blk['lin2_w'].T, lin2=blk['lin2_w'],
            linT=blk['lin_w'].T, lin=blk['lin_w'], bias=bias))
    w1p_l = [jnp.zeros((128, GP), f32).at[:, :G].set(b['mlp_w1']) for b in blocks]
    w1pT_cat = jnp.concatenate([w.T for w in w1p_l], axis=1)
    w1p_cat = jnp.concatenate(w1p_l, axis=1)
    w2T_cat = jnp.concatenate([b['mlp_w2'].T for b in blocks], axis=0)
    w2_cat = jnp.concatenate([b['mlp_w2'] for b in blocks], axis=0)
    biasE = jnp.zeros((8, WD), f32)
    biasE = biasE.at[0, :].set(jnp.concatenate([b['mlp_b1'] for b in blocks]))
    biasE = biasE.at[1, :].set(jnp.concatenate([b['mlp_b2'] for b in blocks]))
    w1rT = params['w1'].T
    w1r = params['w1']
    small = jnp.zeros((8, 64), f32)
    small = small.at[0, :].set(params['b1']).at[1, :].set(params['w2'][0])
    small = small.at[2, :].set(jnp.broadcast_to(params['b2'], (64,)))

    gather_all = _sc_gather_all(NT, EP, CS, ECH)
    msg_fwd = _sc_msg_fwd(NT, NTS, EP, CF)
    msg_bwd = _sc_msg_bwd(NT, NTS, EP, CB)
    msg_bwd0 = _sc_msg_bwd_last(NT, EP, CS)
    scat_force = _sc_scatter_force(NT, NTS, EP, CS)
    edge_geom = _edge_geom(EP)
    ew_fwd = _edge_w_fwd_all(EP, G, GP, NBK)
    ew_bwd = _edge_w_bwd_all(EP, G, GP, NBK)
    node_mm = _node_mm(NT)
    node_fwd = _node_fwd(NT)
    node_bwd1 = _node_bwd1(NT)
    node_bwd2 = _node_bwd2(NT)
    ro_fwd = _readout_fwd(NT)
    ro_bwd = _readout_bwd(NT)
    fcomb = _force_combine(NT)

    h, prow, pcol = gather_all(params['emb'], pos8, z2, rowS, colS)
    d8, ew = edge_geom(prow, pcol)
    Wb = ew_fwd(ew, w1pT_cat, w2T_cat, biasE)

    hxs, aggs = [], []
    for bi in range(NBK):
        t = wT[bi]
        hx = node_mm(h, t['lin1T'])
        hxs.append(hx)
        aggpair = msg_fwd(hx, Wb[bi], rowF, colF)
        h, agg = node_fwd(h, aggpair[0], aggpair[1], t['lin2T'], t['linT'],
                          t['bias'])
        aggs.append(agg)

    out8 = ro_fwd(h, batchp, w1rT, small)
    out = out8[0]

    dh = ro_bwd(h, w1rT, w1r, small)
    dWs = [None] * NBK
    for bi in reversed(range(NBK)):
        t = wT[bi]
        dagg = node_bwd1(dh, aggs[bi], t['lin'], t['lin2T'], t['lin2'],
                         t['bias'])
        if bi > 0:
            dWs[bi], dhxpair = msg_bwd(dagg, hxs[bi], Wb[bi], rowB, colB)
            dh = node_bwd2(dh, dhxpair[0], dhxpair[1], t['lin1'])
        else:
            dWs[bi] = msg_bwd0(dagg, hxs[bi], rowS, colS)

    vec = ew_bwd(ew, dWs[0], dWs[1], dWs[2], dWs[3], d8,
                 w1pT_cat, w1p_cat, w2T_cat, w2_cat, biasE)
    fparts = scat_force(vec, rowS, colS)
    fneg = fcomb(fparts[0, 0], fparts[1, 0], fparts[0, 1], fparts[1, 1])
    forces = fneg[:N, :3]
    return out, forces
